# baseline (speedup 1.0000x reference)
import functools
import math

import jax
import jax.numpy as jnp
from jax import lax
from jax.experimental import pallas as pl
from jax.experimental.pallas import tpu as pltpu
from jax.experimental.pallas import tpu_sc as plsc

F32 = jnp.float32
CHUNK = 128
SC_NC = 2
SC_NS = 16
NW = SC_NC * SC_NS


def _pick_blk(m, target):
    best = None
    for d in range(1, int(math.isqrt(m)) + 1):
        if m % d == 0:
            for c in (d, m // d):
                if c <= target and c % 8 == 0 and (best is None or c > best):
                    best = c
    return best if best is not None else m


def _rowmap(body, row_ins, aux_ins, out_minors, blk_target=8000):
    m = row_ins[0].shape[0]
    blk = _pick_blk(m, blk_target)
    grid = (m // blk,)

    def _rspec(a):
        nd = a.ndim
        return pl.BlockSpec((blk,) + a.shape[1:],
                            lambda i, _nd=nd: (i,) + (0,) * (_nd - 1))

    def _aspec(a):
        nd = a.ndim
        return pl.BlockSpec(a.shape, lambda i, _nd=nd: (0,) * _nd)

    in_specs = [_rspec(a) for a in row_ins] + [_aspec(a) for a in aux_ins]
    out_shape = [jax.ShapeDtypeStruct((m,) + mi, F32) for mi in out_minors]
    out_specs = [pl.BlockSpec((blk,) + mi,
                              lambda i, _nd=len(mi): (i,) + (0,) * _nd)
                 for mi in out_minors]
    outs = pl.pallas_call(
        body, grid=grid, in_specs=in_specs, out_specs=out_specs,
        out_shape=out_shape,
    )(*row_ins, *aux_ins)
    return outs


def _lrelu(x):
    return jnp.maximum(x, 0.01 * x)


def _elu(x):
    return jnp.where(x > 0, x, jnp.exp(jnp.minimum(x, 0.0)) - 1.0)


def _gru_tc(x, h, w):
    i_r = x @ w['ihr'] + w['bihr']
    i_z = x @ w['ihz'] + w['bihz']
    i_n = x @ w['ihn'] + w['bihn']
    h_r = h @ w['hhr'] + w['bhhr']
    h_z = h @ w['hhz'] + w['bhhz']
    h_n = h @ w['hhn'] + w['bhhn']
    r = jax.nn.sigmoid(i_r + h_r)
    z = jax.nn.sigmoid(i_z + h_z)
    n = jnp.tanh(i_n + r * h_n)
    return (1.0 - z) * n + z * h


def _gru_aux(wih, whh, bih, bhh):
    H = wih.shape[1]
    return {
        'ihr': wih[0:H].T, 'ihz': wih[H:2 * H].T, 'ihn': wih[2 * H:].T,
        'hhr': whh[0:H].T, 'hhz': whh[H:2 * H].T, 'hhn': whh[2 * H:].T,
        'bihr': bih[None, 0:H], 'bihz': bih[None, H:2 * H],
        'bihn': bih[None, 2 * H:],
        'bhhr': bhh[None, 0:H], 'bhhz': bhh[None, H:2 * H],
        'bhhn': bhh[None, 2 * H:],
    }


_GRU_KEYS = ('ihr', 'ihz', 'ihn', 'hhr', 'hhz', 'hhn',
             'bihr', 'bihz', 'bihn', 'bhhr', 'bhhz', 'bhhn')


SB = 8
SBR = SB * CHUNK


def _sc_cp():
    return pltpu.CompilerParams(use_tc_tiling_on_sc=False)


def _sc_mesh():
    return plsc.VectorSubcoreMesh(core_axis_name="c", subcore_axis_name="s",
                                  num_cores=SC_NC, num_subcores=SC_NS)


def _rr(nsb, nworkers, wid, fn):
    nfull = nsb // nworkers
    rem = nsb % nworkers
    lax.fori_loop(0, nfull, lambda t, z: (fn(t * nworkers + wid), z)[1], 0)
    if rem:
        @pl.when(wid < rem)
        def _():
            fn(nfull * nworkers + wid)


def _exp_lrelu_block(av, bv, ev, nr):
    for q in range(nr // 16):
        sl = pl.ds(q * 16, 16)
        v = av[sl] + bv[sl]
        ev[sl] = jnp.exp(jnp.maximum(v, 0.01 * v))


def _sc_gather_pair(tab, rtab, src, dst):
    m = src.shape[0]
    nch = m // CHUNK
    nsb, remch = nch // SB, nch % SB
    d = tab.shape[1]
    src2 = src.reshape(nch, CHUNK)
    dst2 = dst.reshape(nch, CHUNK)

    @functools.partial(
        pl.kernel, mesh=_sc_mesh(), compiler_params=_sc_cp(),
        out_type=(jax.ShapeDtypeStruct((m, d), F32),
                  jax.ShapeDtypeStruct((m,), F32)),
        scratch_types=[pltpu.VMEM((SB, CHUNK), jnp.int32),
                       pltpu.VMEM((SB, CHUNK), jnp.int32),
                       pltpu.VMEM((SBR, d), F32),
                       pltpu.VMEM((SBR,), F32),
                       pltpu.SemaphoreType.DMA],
    )
    def k(tab_h, rtab_h, src_h, dst_h, rows_o, rg_o,
          sidx_v, didx_v, rows_v, rv_v, sem):
        wid = lax.axis_index("s") * SC_NC + lax.axis_index("c")

        def sblock(ch0, nj):
            nr = nj * CHUNK
            base = ch0 * CHUNK
            pltpu.sync_copy(src_h.at[pl.ds(ch0, nj)], sidx_v.at[pl.ds(0, nj)])
            pltpu.sync_copy(dst_h.at[pl.ds(ch0, nj)], didx_v.at[pl.ds(0, nj)])
            cps = [pltpu.async_copy(tab_h.at[sidx_v.at[j]],
                                    rows_v.at[pl.ds(j * CHUNK, CHUNK)], sem)
                   for j in range(nj)]
            cps += [pltpu.async_copy(rtab_h.at[didx_v.at[j]],
                                     rv_v.at[pl.ds(j * CHUNK, CHUNK)], sem)
                    for j in range(nj)]
            for cp in cps:
                cp.wait()
            pltpu.sync_copy(rows_v.at[pl.ds(0, nr)],
                            rows_o.at[pl.ds(base, nr)])
            pltpu.sync_copy(rv_v.at[pl.ds(0, nr)], rg_o.at[pl.ds(base, nr)])

        _rr(nsb, NW, wid, lambda sb: sblock(sb * SB, SB))
        if remch:
            @pl.when(wid == NW - 1)
            def _():
                sblock(nsb * SB, remch)

    return k(tab, rtab, src2, dst2)


def _sc_gather_e(tab, atab, btab, src, dst):
    m = src.shape[0]
    nch = m // CHUNK
    nsb, remch = nch // SB, nch % SB
    d = tab.shape[1]
    src2 = src.reshape(nch, CHUNK)
    dst2 = dst.reshape(nch, CHUNK)

    @functools.partial(
        pl.kernel, mesh=_sc_mesh(), compiler_params=_sc_cp(),
        out_type=(jax.ShapeDtypeStruct((m, d), F32),
                  jax.ShapeDtypeStruct((m,), F32)),
        scratch_types=[pltpu.VMEM((SB, CHUNK), jnp.int32),
                       pltpu.VMEM((SB, CHUNK), jnp.int32),
                       pltpu.VMEM((SBR, d), F32),
                       pltpu.VMEM((SBR,), F32),
                       pltpu.VMEM((SBR,), F32),
                       pltpu.VMEM((SBR,), F32),
                       pltpu.SemaphoreType.DMA],
    )
    def k(tab_h, atab_h, btab_h, src_h, dst_h, rows_o, e_o,
          sidx_v, didx_v, rows_v, av, bv, ev, sem):
        wid = lax.axis_index("s") * SC_NC + lax.axis_index("c")

        def sblock(ch0, nj):
            nr = nj * CHUNK
            base = ch0 * CHUNK
            pltpu.sync_copy(src_h.at[pl.ds(ch0, nj)], sidx_v.at[pl.ds(0, nj)])
            pltpu.sync_copy(dst_h.at[pl.ds(ch0, nj)], didx_v.at[pl.ds(0, nj)])
            cps = [pltpu.async_copy(tab_h.at[sidx_v.at[j]],
                                    rows_v.at[pl.ds(j * CHUNK, CHUNK)], sem)
                   for j in range(nj)]
            cps += [pltpu.async_copy(atab_h.at[sidx_v.at[j]],
                                     av.at[pl.ds(j * CHUNK, CHUNK)], sem)
                    for j in range(nj)]
            cps += [pltpu.async_copy(btab_h.at[didx_v.at[j]],
                                     bv.at[pl.ds(j * CHUNK, CHUNK)], sem)
                    for j in range(nj)]
            for cp in cps:
                cp.wait()
            _exp_lrelu_block(av, bv, ev, nr)
            pltpu.sync_copy(rows_v.at[pl.ds(0, nr)],
                            rows_o.at[pl.ds(base, nr)])
            pltpu.sync_copy(ev.at[pl.ds(0, nr)], e_o.at[pl.ds(base, nr)])

        _rr(nsb, NW, wid, lambda sb: sblock(sb * SB, SB))
        if remch:
            @pl.when(wid == NW - 1)
            def _():
                sblock(nsb * SB, remch)

    return k(tab, atab, btab, src2, dst2)


def _sc_scatter_add(vlo, vhi, idx, ew, nrows):
    m = idx.shape[0]
    nch = m // CHUNK
    sbs = 4
    nsb, remch = nch // sbs, nch % sbs
    ra = nrows + 1
    with_d = ew is not None
    idx2 = idx.reshape(nch, CHUNK)

    out_type = [jax.ShapeDtypeStruct((nrows, 32), F32),
                jax.ShapeDtypeStruct((nrows, 32), F32)]
    scratch = [pltpu.VMEM_SHARED((ra, 32), F32),
               pltpu.VMEM((sbs, CHUNK), jnp.int32),
               pltpu.VMEM((sbs * CHUNK, 32), F32),
               pltpu.SemaphoreType.DMA]
    if with_d:
        out_type.append(jax.ShapeDtypeStruct((nrows,), F32))
        scratch += [pltpu.VMEM((sbs * CHUNK,), F32),
                    pltpu.VMEM_SHARED((ra,), F32)]

    zero_u = jnp.zeros((ra, 32), F32)
    zero_d = jnp.zeros((ra,), F32)

    def body(refs):
        if with_d:
            (vlo_h, vhi_h, idx_h, ew_h, zu_h, zd_h, ulo_h, uhi_h, d_h,
             acc_sh, idxb_v, rows_v, sem, ew_v, dacc_sh) = refs
        else:
            (vlo_h, vhi_h, idx_h, zu_h, ulo_h, uhi_h,
             acc_sh, idxb_v, rows_v, sem) = refs
        cid = lax.axis_index("c")
        sid = lax.axis_index("s")

        @pl.when(sid == 0)
        def _():
            pltpu.sync_copy(zu_h, acc_sh)
        if with_d:
            @pl.when(jnp.logical_and(sid == 0, cid == 0))
            def _():
                pltpu.sync_copy(zd_h, dacc_sh)
        plsc.subcore_barrier()

        def sblock(ch0, nj, v_h, inc_d):
            nr = nj * CHUNK
            base = ch0 * CHUNK
            pltpu.sync_copy(idx_h.at[pl.ds(ch0, nj)], idxb_v.at[pl.ds(0, nj)])
            pltpu.sync_copy(v_h.at[pl.ds(base, nr)], rows_v.at[pl.ds(0, nr)])
            if inc_d:
                pltpu.sync_copy(ew_h.at[pl.ds(base, nr)], ew_v.at[pl.ds(0, nr)])
            cps = [pltpu.async_copy(rows_v.at[pl.ds(j * CHUNK, CHUNK)],
                                    acc_sh.at[idxb_v.at[j]], sem, add=True)
                   for j in range(nj)]
            if inc_d:
                cps += [pltpu.async_copy(ew_v.at[pl.ds(j * CHUNK, CHUNK)],
                                         dacc_sh.at[idxb_v.at[j]], sem,
                                         add=True)
                        for j in range(nj)]
            for cp in cps:
                cp.wait()

        def run(v_h, inc_d):
            _rr(nsb, SC_NS, sid,
                lambda sb: sblock(sb * sbs, sbs, v_h, inc_d))
            if remch:
                @pl.when(sid == SC_NS - 1)
                def _():
                    sblock(nsb * sbs, remch, v_h, inc_d)

        @pl.when(cid == 0)
        def _():
            run(vlo_h, with_d)

        @pl.when(cid == 1)
        def _():
            run(vhi_h, False)

        plsc.subcore_barrier()

        @pl.when(jnp.logical_and(sid == 0, cid == 0))
        def _():
            pltpu.sync_copy(acc_sh.at[pl.ds(0, nrows)], ulo_h)

        @pl.when(jnp.logical_and(sid == 0, cid == 1))
        def _():
            pltpu.sync_copy(acc_sh.at[pl.ds(0, nrows)], uhi_h)
        if with_d:
            @pl.when(jnp.logical_and(sid == 0, cid == 0))
            def _():
                pltpu.sync_copy(dacc_sh.at[pl.ds(0, nrows)], d_h)

    if with_d:
        @functools.partial(pl.kernel, mesh=_sc_mesh(), compiler_params=_sc_cp(),
                           out_type=tuple(out_type), scratch_types=scratch)
        def k(*refs):
            body(refs)
        return k(vlo, vhi, idx2, ew, zero_u, zero_d)
    else:
        @functools.partial(pl.kernel, mesh=_sc_mesh(), compiler_params=_sc_cp(),
                           out_type=tuple(out_type), scratch_types=scratch)
        def k(*refs):
            body(refs)
        return k(vlo, vhi, idx2, zero_u)


def _sc_ts_fused(hlo, hhi, t1, t2, bidx, nrows):
    m = bidx.shape[0]
    nch = m // CHUNK
    sbs = 4
    nsb, remch = nch // sbs, nch % sbs
    ra = nrows + 1
    idx2 = bidx.reshape(nch, CHUNK)

    zero_u = jnp.zeros((ra, 32), F32)
    zero_d = jnp.zeros((ra,), F32)

    @functools.partial(
        pl.kernel, mesh=_sc_mesh(), compiler_params=_sc_cp(),
        out_type=(jax.ShapeDtypeStruct((nrows, 32), F32),
                  jax.ShapeDtypeStruct((nrows, 32), F32),
                  jax.ShapeDtypeStruct((nrows,), F32)),
        scratch_types=[pltpu.VMEM_SHARED((ra, 32), F32),
                       pltpu.VMEM_SHARED((ra,), F32),
                       pltpu.VMEM((sbs, CHUNK), jnp.int32),
                       pltpu.VMEM((sbs * CHUNK, 32), F32),
                       pltpu.VMEM((sbs * CHUNK,), F32),
                       pltpu.VMEM((sbs * CHUNK,), F32),
                       pltpu.VMEM((sbs * CHUNK + 16,), F32),
                       pltpu.SemaphoreType.DMA],
    )
    def k(hlo_h, hhi_h, t1_h, t2_h, idx_h, zu_h, zd_h, ulo_h, uhi_h, d_h,
          acc_sh, dacc_sh, idxb_v, rows_v, av, bv, ev, sem):
        cid = lax.axis_index("c")
        sid = lax.axis_index("s")

        @pl.when(sid == 0)
        def _():
            pltpu.sync_copy(zu_h, acc_sh)

        @pl.when(jnp.logical_and(sid == 0, cid == 0))
        def _():
            pltpu.sync_copy(zd_h, dacc_sh)
        plsc.subcore_barrier()

        def sblock(ch0, nj, v_h, inc_d):
            nr = nj * CHUNK
            base = ch0 * CHUNK
            pltpu.sync_copy(idx_h.at[pl.ds(ch0, nj)], idxb_v.at[pl.ds(0, nj)])
            pltpu.sync_copy(v_h.at[pl.ds(base, nr)], rows_v.at[pl.ds(0, nr)])
            pltpu.sync_copy(t1_h.at[pl.ds(base, nr)], av.at[pl.ds(0, nr)])
            cps = [pltpu.async_copy(t2_h.at[idxb_v.at[j]],
                                    bv.at[pl.ds(j * CHUNK, CHUNK)], sem)
                   for j in range(nj)]
            for cp in cps:
                cp.wait()
            _exp_lrelu_block(av, bv, ev, nr)

            def scale(i, z):
                s = ev[pl.ds(i, 16)][0]
                rows_v[i, pl.ds(0, 16)] = rows_v[i, pl.ds(0, 16)] * s
                rows_v[i, pl.ds(16, 16)] = rows_v[i, pl.ds(16, 16)] * s
                return z
            lax.fori_loop(0, nr, scale, 0)
            cps = [pltpu.async_copy(rows_v.at[pl.ds(j * CHUNK, CHUNK)],
                                    acc_sh.at[idxb_v.at[j]], sem, add=True)
                   for j in range(nj)]
            if inc_d:
                cps += [pltpu.async_copy(ev.at[pl.ds(j * CHUNK, CHUNK)],
                                         dacc_sh.at[idxb_v.at[j]], sem,
                                         add=True)
                        for j in range(nj)]
            for cp in cps:
                cp.wait()

        def run(v_h, inc_d):
            _rr(nsb, SC_NS, sid,
                lambda sb: sblock(sb * sbs, sbs, v_h, inc_d))
            if remch:
                @pl.when(sid == SC_NS - 1)
                def _():
                    sblock(nsb * sbs, remch, v_h, inc_d)

        @pl.when(cid == 0)
        def _():
            run(hlo_h, True)

        @pl.when(cid == 1)
        def _():
            run(hhi_h, False)

        plsc.subcore_barrier()

        @pl.when(jnp.logical_and(sid == 0, cid == 0))
        def _():
            pltpu.sync_copy(acc_sh.at[pl.ds(0, nrows)], ulo_h)
            pltpu.sync_copy(dacc_sh.at[pl.ds(0, nrows)], d_h)

        @pl.when(jnp.logical_and(sid == 0, cid == 1))
        def _():
            pltpu.sync_copy(acc_sh.at[pl.ds(0, nrows)], uhi_h)

    return k(hlo, hhi, t1, t2, idx2, zero_u, zero_d)


def _pad_rows(a, mult=CHUNK, value=0):
    m = a.shape[0]
    pad = (-m) % mult
    if pad == 0:
        return a
    cfg = [(0, pad)] + [(0, 0)] * (a.ndim - 1)
    return jnp.pad(a, cfg, constant_values=value)


def _t_h0(x_r, lin1T, lin1b, attr, h0_r, r_r):
    h0 = _lrelu(x_r[...] @ lin1T[...] + lin1b[...])
    h0_r[...] = h0
    r_r[...] = h0 @ attr[...]


def _t_gate(xj_r, ea_r, rg_r, w1aT, w1bT, attl, slo_r, shi_r, e1_r):
    m = _lrelu(xj_r[...] @ w1aT[...] + ea_r[...] @ w1bT[...])
    ma = m @ attl[...]
    e = jnp.exp(_lrelu(ma + rg_r[...]))
    s = e * m
    slo_r[...] = s[:, :32]
    shi_r[...] = s[:, 32:]
    e1_r[...] = e


def _t_scale_edge(xjh_r, e_r, slo_r, shi_r, e1_r):
    e = e_r[...]
    s = e * xjh_r[...]
    slo_r[...] = s[:, :32]
    shi_r[...] = s[:, 32:]
    e1_r[...] = e


def _make_t_nodeupd(project, nxt):
    def body(*refs):
        (ulo_r, uhi_r, d_r, h_r) = refs[:4]
        i = 4
        if project:
            lin2T, bias = refs[i], refs[i + 1]
            i += 2
        else:
            bias = refs[i]
            i += 1
        gru = dict(zip(_GRU_KEYS, refs[i:i + 12]))
        i += 12
        if nxt:
            wT, asrc, adst = refs[i], refs[i + 1], refs[i + 2]
            i += 3
        outs = refs[i:]
        u = jnp.concatenate([ulo_r[...], uhi_r[...]], axis=1)
        agg = u / (d_r[...] + 1e-16)
        if project:
            agg = agg @ lin2T[...] + bias[...]
        else:
            agg = agg + bias[...]
        hcand = _elu(agg)
        gw = {kk: vv[...] for kk, vv in gru.items()}
        xc = jnp.maximum(_gru_tc(hcand, h_r[...], gw), 0.0)
        outs[0][...] = xc[:, :32]
        outs[1][...] = xc[:, 32:]
        if nxt:
            hs = xc @ wT[...]
            outs[2][...] = hs
            outs[3][...] = hs @ asrc[...]
            outs[4][...] = hs @ adst[...]
    return body


def _t_ts_hoist(xlo_r, xhi_r, molT, asrc, hlo_r, hhi_r, t1_r):
    xc = jnp.concatenate([xlo_r[...], xhi_r[...]], axis=1)
    hs = xc @ molT[...]
    hlo_r[...] = hs[:, :32]
    hhi_r[...] = hs[:, 32:]
    t1_r[...] = hs @ asrc[...]


def _t_read_t2(ulo_r, uhi_r, molT, adst, out_r, t2_r):
    out = jnp.maximum(
        jnp.concatenate([ulo_r[...], uhi_r[...]], axis=1), 0.0)
    out_r[...] = out
    t2_r[...] = (out @ molT[...]) @ adst[...]


def _t_ts_post_t2(*refs):
    ulo_r, uhi_r, d_r, out_r = refs[:4]
    bias = refs[4]
    gru = dict(zip(_GRU_KEYS, refs[5:17]))
    molT, adst = refs[17], refs[18]
    newout_r, t2_r = refs[19], refs[20]
    u = jnp.concatenate([ulo_r[...], uhi_r[...]], axis=1)
    h = _elu(u / (d_r[...] + 1e-16) + bias[...])
    gw = {kk: vv[...] for kk, vv in gru.items()}
    out = jnp.maximum(_gru_tc(h, out_r[...], gw), 0.0)
    newout_r[...] = out
    t2_r[...] = (out @ molT[...]) @ adst[...]


_BNF = 1.0 / math.sqrt(1.0 + 1e-5)


def _t_fd(fp_r, de_r, fw1T, fb1, fg, fbb, fw2T, fb2,
          dw1T, db1, dg, dbb, dw2T, db2, f_r, d_r):
    f = jnp.maximum(fp_r[...] @ fw1T[...] + fb1[...], 0.0)
    f = f * _BNF * fg[...] + fbb[...]
    f_r[...] = jnp.maximum(f @ fw2T[...] + fb2[...], 0.0)
    d = jnp.maximum(de_r[...] @ dw1T[...] + db1[...], 0.0)
    d = d * _BNF * dg[...] + dbb[...]
    d_r[...] = jnp.maximum(d @ dw2T[...] + db2[...], 0.0)


def _t_final(out_r, f_r, d_r, lin2T, lin2b, fusT, fusb,
             toxT, toxb, regT, regb, logits_r, pct_r):
    ge = out_r[...] @ lin2T[...] + lin2b[...]
    comb = jnp.concatenate([ge, f_r[...], d_r[...]], axis=1)
    shared = jnp.maximum(comb @ fusT[...] + fusb[...], 0.0)
    logits_r[...] = shared @ toxT[...] + toxb[...]
    pct_r[...] = shared @ regT[...] + regb[...]


def kernel(x, edge_index, edge_attr, batch, fingerprints, descriptors, params):
    p = params
    n = x.shape[0]
    b = fingerprints.shape[0]
    n_layers_m1 = p['atom_lin_w'].shape[0]

    src = edge_index[0].astype(jnp.int32)
    dst = edge_index[1].astype(jnp.int32)
    batch = batch.astype(jnp.int32)

    f_mlp, d_mlp = _rowmap(
        _t_fd, [fingerprints, descriptors],
        [p['fp_w1'].T, p['fp_b1'][None, :], p['fp_bn_g'][None, :],
         p['fp_bn_b'][None, :], p['fp_w2'].T, p['fp_b2'][None, :],
         p['desc_w1'].T, p['desc_b1'][None, :], p['desc_bn_g'][None, :],
         p['desc_bn_b'][None, :], p['desc_w2'].T, p['desc_b2'][None, :]],
        [(64,), (64,)], blk_target=1024)

    h0, r = _rowmap(_t_h0, [x],
                    [p['lin1_w'].T, p['lin1_b'][None, :],
                     p['gate_att_r'][:, None]],
                    [(64,), (1,)])

    xj, rg = _sc_gather_pair(h0, r[:, 0], src, dst)
    slo, shi, e1 = _rowmap(
        _t_gate, [xj, edge_attr, rg[:, None]],
        [p['gate_lin1_w'][:, :64].T, p['gate_lin1_w'][:, 64:].T,
         p['gate_att_l'][:, None]],
        [(32,), (32,), (1,)])
    ulo, uhi, dsum = _sc_scatter_add(slo, shi, dst, e1[:, 0], n)

    def _atom_pre_aux(l):
        return [p['atom_lin_w'][l].T, p['atom_att_src'][l][:, None],
                p['atom_att_dst'][l][:, None]]

    gate_aux = [p['gate_lin2_w'].T, p['gate_bias'][None, :]]
    gru0 = _gru_aux(p['gru0_wih'], p['gru0_whh'], p['gru0_bih'], p['gru0_bhh'])
    gate_aux += [gru0[kk] for kk in _GRU_KEYS]
    xlo, xhi, hs, a_s, a_d = _rowmap(
        _make_t_nodeupd(True, True),
        [ulo, uhi, dsum[:, None], h0], gate_aux + _atom_pre_aux(0),
        [(32,), (32,), (64,), (1,), (1,)])

    for l in range(n_layers_m1):
        xjh, ev = _sc_gather_e(hs, a_s[:, 0], a_d[:, 0], src, dst)
        slo, shi, e1 = _rowmap(_t_scale_edge,
                               [xjh, ev[:, None]], [],
                               [(32,), (32,), (1,)])
        ulo, uhi, dsum = _sc_scatter_add(slo, shi, dst, e1[:, 0], n)
        aux = [p['atom_bias'][l][None, :]]
        grul = _gru_aux(p['atom_gru_wih'][l], p['atom_gru_whh'][l],
                        p['atom_gru_bih'][l], p['atom_gru_bhh'][l])
        aux += [grul[kk] for kk in _GRU_KEYS]
        xc_prev = jnp.concatenate([xlo, xhi], axis=1)
        last = l == n_layers_m1 - 1
        if not last:
            xlo, xhi, hs, a_s, a_d = _rowmap(
                _make_t_nodeupd(False, True),
                [ulo, uhi, dsum[:, None], xc_prev],
                aux + _atom_pre_aux(l + 1),
                [(32,), (32,), (64,), (1,), (1,)])
        else:
            xlo, xhi = _rowmap(_make_t_nodeupd(False, False),
                               [ulo, uhi, dsum[:, None], xc_prev], aux,
                               [(32,), (32,)])

    batch_pad = _pad_rows(batch, value=b)
    ulo, uhi = _sc_scatter_add(_pad_rows(xlo), _pad_rows(xhi),
                               batch_pad, None, b)
    out, t2 = _rowmap(_t_read_t2, [ulo, uhi],
                      [p['mol_lin_w'].T, p['mol_att_dst'][:, None]],
                      [(64,), (1,)], blk_target=1024)

    hlo, hhi, t1 = _rowmap(
        _t_ts_hoist, [xlo, xhi],
        [p['mol_lin_w'].T, p['mol_att_src'][:, None]],
        [(32,), (32,), (1,)])
    hlo_p, hhi_p, t1_p = (_pad_rows(hlo), _pad_rows(hhi),
                          _pad_rows(t1[:, 0]))

    molgru = _gru_aux(p['molgru_wih'], p['molgru_whh'],
                      p['molgru_bih'], p['molgru_bhh'])
    ts_post_aux = ([p['mol_bias'][None, :]]
                   + [molgru[kk] for kk in _GRU_KEYS]
                   + [p['mol_lin_w'].T, p['mol_att_dst'][:, None]])
    for _ in range(3):
        ulo, uhi, dsum = _sc_ts_fused(hlo_p, hhi_p, t1_p, t2[:, 0],
                                      batch_pad, b)
        out, t2 = _rowmap(_t_ts_post_t2,
                          [ulo, uhi, dsum[:, None], out], ts_post_aux,
                          [(64,), (1,)], blk_target=1024)

    logits, pct = _rowmap(
        _t_final, [out, f_mlp, d_mlp],
        [p['lin2_w'].T, p['lin2_b'][None, :], p['fus_w'].T,
         p['fus_b'][None, :], p['tox_w'].T, p['tox_b'][None, :],
         p['reg_w'].T, p['reg_b'][None, :]],
        [(13,), (1,)], blk_target=1024)
    return logits, pct[:, 0]


# speedup vs baseline: 1.0267x; 1.0267x over previous
import functools
import math

import jax
import jax.numpy as jnp
from jax import lax
from jax.experimental import pallas as pl
from jax.experimental.pallas import tpu as pltpu
from jax.experimental.pallas import tpu_sc as plsc

F32 = jnp.float32
CHUNK = 128
SC_NC = 2
SC_NS = 16
NW = SC_NC * SC_NS


def _pick_blk(m, target):
    best = None
    for d in range(1, int(math.isqrt(m)) + 1):
        if m % d == 0:
            for c in (d, m // d):
                if c <= target and c % 8 == 0 and (best is None or c > best):
                    best = c
    return best if best is not None else m


def _rowmap(body, row_ins, aux_ins, out_minors, blk_target=8000):
    m = row_ins[0].shape[0]
    blk = _pick_blk(m, blk_target)
    grid = (m // blk,)

    def _rspec(a):
        nd = a.ndim
        return pl.BlockSpec((blk,) + a.shape[1:],
                            lambda i, _nd=nd: (i,) + (0,) * (_nd - 1))

    def _aspec(a):
        nd = a.ndim
        return pl.BlockSpec(a.shape, lambda i, _nd=nd: (0,) * _nd)

    in_specs = [_rspec(a) for a in row_ins] + [_aspec(a) for a in aux_ins]
    out_shape = [jax.ShapeDtypeStruct((m,) + mi, F32) for mi in out_minors]
    out_specs = [pl.BlockSpec((blk,) + mi,
                              lambda i, _nd=len(mi): (i,) + (0,) * _nd)
                 for mi in out_minors]
    outs = pl.pallas_call(
        body, grid=grid, in_specs=in_specs, out_specs=out_specs,
        out_shape=out_shape,
    )(*row_ins, *aux_ins)
    return outs


def _lrelu(x):
    return jnp.maximum(x, 0.01 * x)


def _elu(x):
    return jnp.where(x > 0, x, jnp.exp(jnp.minimum(x, 0.0)) - 1.0)


def _gru_tc(x, h, w):
    i_r = x @ w['ihr'] + w['bihr']
    i_z = x @ w['ihz'] + w['bihz']
    i_n = x @ w['ihn'] + w['bihn']
    h_r = h @ w['hhr'] + w['bhhr']
    h_z = h @ w['hhz'] + w['bhhz']
    h_n = h @ w['hhn'] + w['bhhn']
    r = jax.nn.sigmoid(i_r + h_r)
    z = jax.nn.sigmoid(i_z + h_z)
    n = jnp.tanh(i_n + r * h_n)
    return (1.0 - z) * n + z * h


def _gru_aux(wih, whh, bih, bhh):
    H = wih.shape[1]
    return {
        'ihr': wih[0:H].T, 'ihz': wih[H:2 * H].T, 'ihn': wih[2 * H:].T,
        'hhr': whh[0:H].T, 'hhz': whh[H:2 * H].T, 'hhn': whh[2 * H:].T,
        'bihr': bih[None, 0:H], 'bihz': bih[None, H:2 * H],
        'bihn': bih[None, 2 * H:],
        'bhhr': bhh[None, 0:H], 'bhhz': bhh[None, H:2 * H],
        'bhhn': bhh[None, 2 * H:],
    }


_GRU_KEYS = ('ihr', 'ihz', 'ihn', 'hhr', 'hhz', 'hhn',
             'bihr', 'bihz', 'bihn', 'bhhr', 'bhhz', 'bhhn')


SB = 12
SBR = SB * CHUNK


def _sc_cp():
    return pltpu.CompilerParams(use_tc_tiling_on_sc=False)


def _sc_mesh():
    return plsc.VectorSubcoreMesh(core_axis_name="c", subcore_axis_name="s",
                                  num_cores=SC_NC, num_subcores=SC_NS)


def _rr(nsb, nworkers, wid, fn):
    nfull = nsb // nworkers
    rem = nsb % nworkers
    lax.fori_loop(0, nfull, lambda t, z: (fn(t * nworkers + wid), z)[1], 0)
    if rem:
        @pl.when(wid < rem)
        def _():
            fn(nfull * nworkers + wid)


def _exp_lrelu_block(av, bv, ev, nr):
    for q in range(nr // 16):
        sl = pl.ds(q * 16, 16)
        v = av[sl] + bv[sl]
        ev[sl] = jnp.exp(jnp.maximum(v, 0.01 * v))


def _sc_gather_pair(tab, rtab, src, dst):
    m = src.shape[0]
    nch = m // CHUNK
    nsb, remch = nch // SB, nch % SB
    d = tab.shape[1]
    src2 = src.reshape(nch, CHUNK)
    dst2 = dst.reshape(nch, CHUNK)

    @functools.partial(
        pl.kernel, mesh=_sc_mesh(), compiler_params=_sc_cp(),
        out_type=(jax.ShapeDtypeStruct((m, d), F32),
                  jax.ShapeDtypeStruct((m,), F32)),
        scratch_types=[pltpu.VMEM((SB, CHUNK), jnp.int32),
                       pltpu.VMEM((SB, CHUNK), jnp.int32),
                       pltpu.VMEM((SBR, d), F32),
                       pltpu.VMEM((SBR,), F32),
                       pltpu.SemaphoreType.DMA],
    )
    def k(tab_h, rtab_h, src_h, dst_h, rows_o, rg_o,
          sidx_v, didx_v, rows_v, rv_v, sem):
        wid = lax.axis_index("s") * SC_NC + lax.axis_index("c")

        def sblock(ch0, nj):
            nr = nj * CHUNK
            base = ch0 * CHUNK
            pltpu.sync_copy(src_h.at[pl.ds(ch0, nj)], sidx_v.at[pl.ds(0, nj)])
            pltpu.sync_copy(dst_h.at[pl.ds(ch0, nj)], didx_v.at[pl.ds(0, nj)])
            cps = [pltpu.async_copy(tab_h.at[sidx_v.at[j]],
                                    rows_v.at[pl.ds(j * CHUNK, CHUNK)], sem)
                   for j in range(nj)]
            cps += [pltpu.async_copy(rtab_h.at[didx_v.at[j]],
                                     rv_v.at[pl.ds(j * CHUNK, CHUNK)], sem)
                    for j in range(nj)]
            for cp in cps:
                cp.wait()
            pltpu.sync_copy(rows_v.at[pl.ds(0, nr)],
                            rows_o.at[pl.ds(base, nr)])
            pltpu.sync_copy(rv_v.at[pl.ds(0, nr)], rg_o.at[pl.ds(base, nr)])

        _rr(nsb, NW, wid, lambda sb: sblock(sb * SB, SB))
        if remch:
            @pl.when(wid == NW - 1)
            def _():
                sblock(nsb * SB, remch)

    return k(tab, rtab, src2, dst2)


def _sc_gather_e(tab, atab, btab, src, dst):
    m = src.shape[0]
    nch = m // CHUNK
    nsb, remch = nch // SB, nch % SB
    d = tab.shape[1]
    src2 = src.reshape(nch, CHUNK)
    dst2 = dst.reshape(nch, CHUNK)

    @functools.partial(
        pl.kernel, mesh=_sc_mesh(), compiler_params=_sc_cp(),
        out_type=(jax.ShapeDtypeStruct((m, d), F32),
                  jax.ShapeDtypeStruct((m,), F32)),
        scratch_types=[pltpu.VMEM((SB, CHUNK), jnp.int32),
                       pltpu.VMEM((SB, CHUNK), jnp.int32),
                       pltpu.VMEM((SBR, d), F32),
                       pltpu.VMEM((SBR,), F32),
                       pltpu.VMEM((SBR,), F32),
                       pltpu.VMEM((SBR,), F32),
                       pltpu.SemaphoreType.DMA],
    )
    def k(tab_h, atab_h, btab_h, src_h, dst_h, rows_o, e_o,
          sidx_v, didx_v, rows_v, av, bv, ev, sem):
        wid = lax.axis_index("s") * SC_NC + lax.axis_index("c")

        def sblock(ch0, nj):
            nr = nj * CHUNK
            base = ch0 * CHUNK
            pltpu.sync_copy(src_h.at[pl.ds(ch0, nj)], sidx_v.at[pl.ds(0, nj)])
            pltpu.sync_copy(dst_h.at[pl.ds(ch0, nj)], didx_v.at[pl.ds(0, nj)])
            cps = [pltpu.async_copy(tab_h.at[sidx_v.at[j]],
                                    rows_v.at[pl.ds(j * CHUNK, CHUNK)], sem)
                   for j in range(nj)]
            cps += [pltpu.async_copy(atab_h.at[sidx_v.at[j]],
                                     av.at[pl.ds(j * CHUNK, CHUNK)], sem)
                    for j in range(nj)]
            cps += [pltpu.async_copy(btab_h.at[didx_v.at[j]],
                                     bv.at[pl.ds(j * CHUNK, CHUNK)], sem)
                    for j in range(nj)]
            for cp in cps:
                cp.wait()
            _exp_lrelu_block(av, bv, ev, nr)
            pltpu.sync_copy(rows_v.at[pl.ds(0, nr)],
                            rows_o.at[pl.ds(base, nr)])
            pltpu.sync_copy(ev.at[pl.ds(0, nr)], e_o.at[pl.ds(base, nr)])

        _rr(nsb, NW, wid, lambda sb: sblock(sb * SB, SB))
        if remch:
            @pl.when(wid == NW - 1)
            def _():
                sblock(nsb * SB, remch)

    return k(tab, atab, btab, src2, dst2)


def _sc_scatter_add(vlo, vhi, idx, ew, nrows):
    m = idx.shape[0]
    nch = m // CHUNK
    sbs = 6
    nsb, remch = nch // sbs, nch % sbs
    ra = nrows + 1
    with_d = ew is not None
    idx2 = idx.reshape(nch, CHUNK)

    out_type = [jax.ShapeDtypeStruct((nrows, 32), F32),
                jax.ShapeDtypeStruct((nrows, 32), F32)]
    scratch = [pltpu.VMEM_SHARED((ra, 32), F32),
               pltpu.VMEM((sbs, CHUNK), jnp.int32),
               pltpu.VMEM((sbs * CHUNK, 32), F32),
               pltpu.SemaphoreType.DMA]
    if with_d:
        out_type.append(jax.ShapeDtypeStruct((nrows,), F32))
        scratch += [pltpu.VMEM((sbs * CHUNK,), F32),
                    pltpu.VMEM_SHARED((ra,), F32)]

    zero_u = jnp.zeros((ra, 32), F32)
    zero_d = jnp.zeros((ra,), F32)

    def body(refs):
        if with_d:
            (vlo_h, vhi_h, idx_h, ew_h, zu_h, zd_h, ulo_h, uhi_h, d_h,
             acc_sh, idxb_v, rows_v, sem, ew_v, dacc_sh) = refs
        else:
            (vlo_h, vhi_h, idx_h, zu_h, ulo_h, uhi_h,
             acc_sh, idxb_v, rows_v, sem) = refs
        cid = lax.axis_index("c")
        sid = lax.axis_index("s")

        @pl.when(sid == 0)
        def _():
            pltpu.sync_copy(zu_h, acc_sh)
        if with_d:
            @pl.when(jnp.logical_and(sid == 0, cid == 0))
            def _():
                pltpu.sync_copy(zd_h, dacc_sh)
        plsc.subcore_barrier()

        def sblock(ch0, nj, v_h, inc_d):
            nr = nj * CHUNK
            base = ch0 * CHUNK
            pltpu.sync_copy(idx_h.at[pl.ds(ch0, nj)], idxb_v.at[pl.ds(0, nj)])
            pltpu.sync_copy(v_h.at[pl.ds(base, nr)], rows_v.at[pl.ds(0, nr)])
            if inc_d:
                pltpu.sync_copy(ew_h.at[pl.ds(base, nr)], ew_v.at[pl.ds(0, nr)])
            cps = [pltpu.async_copy(rows_v.at[pl.ds(j * CHUNK, CHUNK)],
                                    acc_sh.at[idxb_v.at[j]], sem, add=True)
                   for j in range(nj)]
            if inc_d:
                cps += [pltpu.async_copy(ew_v.at[pl.ds(j * CHUNK, CHUNK)],
                                         dacc_sh.at[idxb_v.at[j]], sem,
                                         add=True)
                        for j in range(nj)]
            for cp in cps:
                cp.wait()

        def run(v_h, inc_d):
            _rr(nsb, SC_NS, sid,
                lambda sb: sblock(sb * sbs, sbs, v_h, inc_d))
            if remch:
                @pl.when(sid == SC_NS - 1)
                def _():
                    sblock(nsb * sbs, remch, v_h, inc_d)

        @pl.when(cid == 0)
        def _():
            run(vlo_h, with_d)

        @pl.when(cid == 1)
        def _():
            run(vhi_h, False)

        plsc.subcore_barrier()

        @pl.when(jnp.logical_and(sid == 0, cid == 0))
        def _():
            pltpu.sync_copy(acc_sh.at[pl.ds(0, nrows)], ulo_h)

        @pl.when(jnp.logical_and(sid == 0, cid == 1))
        def _():
            pltpu.sync_copy(acc_sh.at[pl.ds(0, nrows)], uhi_h)
        if with_d:
            @pl.when(jnp.logical_and(sid == 0, cid == 0))
            def _():
                pltpu.sync_copy(dacc_sh.at[pl.ds(0, nrows)], d_h)

    if with_d:
        @functools.partial(pl.kernel, mesh=_sc_mesh(), compiler_params=_sc_cp(),
                           out_type=tuple(out_type), scratch_types=scratch)
        def k(*refs):
            body(refs)
        return k(vlo, vhi, idx2, ew, zero_u, zero_d)
    else:
        @functools.partial(pl.kernel, mesh=_sc_mesh(), compiler_params=_sc_cp(),
                           out_type=tuple(out_type), scratch_types=scratch)
        def k(*refs):
            body(refs)
        return k(vlo, vhi, idx2, zero_u)


def _sc_ts_fused(hlo, hhi, t1, t2, bidx, nrows):
    m = bidx.shape[0]
    nch = m // CHUNK
    sbs = 8
    nsb, remch = nch // sbs, nch % sbs
    ra = nrows + 1
    idx2 = bidx.reshape(nch, CHUNK)

    zero_u = jnp.zeros((ra, 32), F32)
    zero_d = jnp.zeros((ra,), F32)

    @functools.partial(
        pl.kernel, mesh=_sc_mesh(), compiler_params=_sc_cp(),
        out_type=(jax.ShapeDtypeStruct((nrows, 32), F32),
                  jax.ShapeDtypeStruct((nrows, 32), F32),
                  jax.ShapeDtypeStruct((nrows,), F32)),
        scratch_types=[pltpu.VMEM_SHARED((ra, 32), F32),
                       pltpu.VMEM_SHARED((ra,), F32),
                       pltpu.VMEM((sbs, CHUNK), jnp.int32),
                       pltpu.VMEM((sbs * CHUNK, 32), F32),
                       pltpu.VMEM((sbs * CHUNK,), F32),
                       pltpu.VMEM((sbs * CHUNK,), F32),
                       pltpu.VMEM((sbs * CHUNK + 16,), F32),
                       pltpu.SemaphoreType.DMA],
    )
    def k(hlo_h, hhi_h, t1_h, t2_h, idx_h, zu_h, zd_h, ulo_h, uhi_h, d_h,
          acc_sh, dacc_sh, idxb_v, rows_v, av, bv, ev, sem):
        cid = lax.axis_index("c")
        sid = lax.axis_index("s")

        @pl.when(sid == 0)
        def _():
            pltpu.sync_copy(zu_h, acc_sh)

        @pl.when(jnp.logical_and(sid == 0, cid == 0))
        def _():
            pltpu.sync_copy(zd_h, dacc_sh)
        plsc.subcore_barrier()

        def sblock(ch0, nj, v_h, inc_d):
            nr = nj * CHUNK
            base = ch0 * CHUNK
            pltpu.sync_copy(idx_h.at[pl.ds(ch0, nj)], idxb_v.at[pl.ds(0, nj)])
            pltpu.sync_copy(v_h.at[pl.ds(base, nr)], rows_v.at[pl.ds(0, nr)])
            pltpu.sync_copy(t1_h.at[pl.ds(base, nr)], av.at[pl.ds(0, nr)])
            cps = [pltpu.async_copy(t2_h.at[idxb_v.at[j]],
                                    bv.at[pl.ds(j * CHUNK, CHUNK)], sem)
                   for j in range(nj)]
            for cp in cps:
                cp.wait()
            _exp_lrelu_block(av, bv, ev, nr)

            def scale(i, z):
                s = ev[pl.ds(i, 16)][0]
                rows_v[i, pl.ds(0, 16)] = rows_v[i, pl.ds(0, 16)] * s
                rows_v[i, pl.ds(16, 16)] = rows_v[i, pl.ds(16, 16)] * s
                return z
            lax.fori_loop(0, nr, scale, 0)
            cps = [pltpu.async_copy(rows_v.at[pl.ds(j * CHUNK, CHUNK)],
                                    acc_sh.at[idxb_v.at[j]], sem, add=True)
                   for j in range(nj)]
            if inc_d:
                cps += [pltpu.async_copy(ev.at[pl.ds(j * CHUNK, CHUNK)],
                                         dacc_sh.at[idxb_v.at[j]], sem,
                                         add=True)
                        for j in range(nj)]
            for cp in cps:
                cp.wait()

        def run(v_h, inc_d):
            _rr(nsb, SC_NS, sid,
                lambda sb: sblock(sb * sbs, sbs, v_h, inc_d))
            if remch:
                @pl.when(sid == SC_NS - 1)
                def _():
                    sblock(nsb * sbs, remch, v_h, inc_d)

        @pl.when(cid == 0)
        def _():
            run(hlo_h, True)

        @pl.when(cid == 1)
        def _():
            run(hhi_h, False)

        plsc.subcore_barrier()

        @pl.when(jnp.logical_and(sid == 0, cid == 0))
        def _():
            pltpu.sync_copy(acc_sh.at[pl.ds(0, nrows)], ulo_h)
            pltpu.sync_copy(dacc_sh.at[pl.ds(0, nrows)], d_h)

        @pl.when(jnp.logical_and(sid == 0, cid == 1))
        def _():
            pltpu.sync_copy(acc_sh.at[pl.ds(0, nrows)], uhi_h)

    return k(hlo, hhi, t1, t2, idx2, zero_u, zero_d)


def _pad_rows(a, mult=CHUNK, value=0):
    m = a.shape[0]
    pad = (-m) % mult
    if pad == 0:
        return a
    cfg = [(0, pad)] + [(0, 0)] * (a.ndim - 1)
    return jnp.pad(a, cfg, constant_values=value)


def _t_h0(x_r, lin1T, lin1b, attr, h0_r, r_r):
    h0 = _lrelu(x_r[...] @ lin1T[...] + lin1b[...])
    h0_r[...] = h0
    r_r[...] = h0 @ attr[...]


def _t_gate(xj_r, ea_r, rg_r, w1aT, w1bT, attl, slo_r, shi_r, e1_r):
    m = _lrelu(xj_r[...] @ w1aT[...] + ea_r[...] @ w1bT[...])
    ma = m @ attl[...]
    e = jnp.exp(_lrelu(ma + rg_r[...]))
    s = e * m
    slo_r[...] = s[:, :32]
    shi_r[...] = s[:, 32:]
    e1_r[...] = e


def _t_scale_edge(xjh_r, e_r, slo_r, shi_r, e1_r):
    e = e_r[...]
    s = e * xjh_r[...]
    slo_r[...] = s[:, :32]
    shi_r[...] = s[:, 32:]
    e1_r[...] = e


def _make_t_nodeupd(project, nxt):
    def body(*refs):
        (ulo_r, uhi_r, d_r, h_r) = refs[:4]
        i = 4
        if project:
            lin2T, bias = refs[i], refs[i + 1]
            i += 2
        else:
            bias = refs[i]
            i += 1
        gru = dict(zip(_GRU_KEYS, refs[i:i + 12]))
        i += 12
        if nxt:
            wT, asrc, adst = refs[i], refs[i + 1], refs[i + 2]
            i += 3
        outs = refs[i:]
        u = jnp.concatenate([ulo_r[...], uhi_r[...]], axis=1)
        agg = u / (d_r[...] + 1e-16)
        if project:
            agg = agg @ lin2T[...] + bias[...]
        else:
            agg = agg + bias[...]
        hcand = _elu(agg)
        gw = {kk: vv[...] for kk, vv in gru.items()}
        xc = jnp.maximum(_gru_tc(hcand, h_r[...], gw), 0.0)
        outs[0][...] = xc[:, :32]
        outs[1][...] = xc[:, 32:]
        if nxt:
            hs = xc @ wT[...]
            outs[2][...] = hs
            outs[3][...] = hs @ asrc[...]
            outs[4][...] = hs @ adst[...]
    return body


def _t_ts_hoist(xlo_r, xhi_r, molT, asrc, hlo_r, hhi_r, t1_r):
    xc = jnp.concatenate([xlo_r[...], xhi_r[...]], axis=1)
    hs = xc @ molT[...]
    hlo_r[...] = hs[:, :32]
    hhi_r[...] = hs[:, 32:]
    t1_r[...] = hs @ asrc[...]


def _t_read_t2(ulo_r, uhi_r, molT, adst, out_r, t2_r):
    out = jnp.maximum(
        jnp.concatenate([ulo_r[...], uhi_r[...]], axis=1), 0.0)
    out_r[...] = out
    t2_r[...] = (out @ molT[...]) @ adst[...]


def _t_ts_post_t2(*refs):
    ulo_r, uhi_r, d_r, out_r = refs[:4]
    bias = refs[4]
    gru = dict(zip(_GRU_KEYS, refs[5:17]))
    molT, adst = refs[17], refs[18]
    newout_r, t2_r = refs[19], refs[20]
    u = jnp.concatenate([ulo_r[...], uhi_r[...]], axis=1)
    h = _elu(u / (d_r[...] + 1e-16) + bias[...])
    gw = {kk: vv[...] for kk, vv in gru.items()}
    out = jnp.maximum(_gru_tc(h, out_r[...], gw), 0.0)
    newout_r[...] = out
    t2_r[...] = (out @ molT[...]) @ adst[...]


_BNF = 1.0 / math.sqrt(1.0 + 1e-5)


def _t_fd(fp_r, de_r, fw1T, fb1, fg, fbb, fw2T, fb2,
          dw1T, db1, dg, dbb, dw2T, db2, f_r, d_r):
    f = jnp.maximum(fp_r[...] @ fw1T[...] + fb1[...], 0.0)
    f = f * _BNF * fg[...] + fbb[...]
    f_r[...] = jnp.maximum(f @ fw2T[...] + fb2[...], 0.0)
    d = jnp.maximum(de_r[...] @ dw1T[...] + db1[...], 0.0)
    d = d * _BNF * dg[...] + dbb[...]
    d_r[...] = jnp.maximum(d @ dw2T[...] + db2[...], 0.0)


def _t_final(out_r, f_r, d_r, lin2T, lin2b, fusT, fusb,
             toxT, toxb, regT, regb, logits_r, pct_r):
    ge = out_r[...] @ lin2T[...] + lin2b[...]
    comb = jnp.concatenate([ge, f_r[...], d_r[...]], axis=1)
    shared = jnp.maximum(comb @ fusT[...] + fusb[...], 0.0)
    logits_r[...] = shared @ toxT[...] + toxb[...]
    pct_r[...] = shared @ regT[...] + regb[...]


def kernel(x, edge_index, edge_attr, batch, fingerprints, descriptors, params):
    p = params
    n = x.shape[0]
    b = fingerprints.shape[0]
    n_layers_m1 = p['atom_lin_w'].shape[0]

    src = edge_index[0].astype(jnp.int32)
    dst = edge_index[1].astype(jnp.int32)
    batch = batch.astype(jnp.int32)

    f_mlp, d_mlp = _rowmap(
        _t_fd, [fingerprints, descriptors],
        [p['fp_w1'].T, p['fp_b1'][None, :], p['fp_bn_g'][None, :],
         p['fp_bn_b'][None, :], p['fp_w2'].T, p['fp_b2'][None, :],
         p['desc_w1'].T, p['desc_b1'][None, :], p['desc_bn_g'][None, :],
         p['desc_bn_b'][None, :], p['desc_w2'].T, p['desc_b2'][None, :]],
        [(64,), (64,)], blk_target=1024)

    h0, r = _rowmap(_t_h0, [x],
                    [p['lin1_w'].T, p['lin1_b'][None, :],
                     p['gate_att_r'][:, None]],
                    [(64,), (1,)])

    xj, rg = _sc_gather_pair(h0, r[:, 0], src, dst)
    slo, shi, e1 = _rowmap(
        _t_gate, [xj, edge_attr, rg[:, None]],
        [p['gate_lin1_w'][:, :64].T, p['gate_lin1_w'][:, 64:].T,
         p['gate_att_l'][:, None]],
        [(32,), (32,), (1,)])
    ulo, uhi, dsum = _sc_scatter_add(slo, shi, dst, e1[:, 0], n)

    def _atom_pre_aux(l):
        return [p['atom_lin_w'][l].T, p['atom_att_src'][l][:, None],
                p['atom_att_dst'][l][:, None]]

    gate_aux = [p['gate_lin2_w'].T, p['gate_bias'][None, :]]
    gru0 = _gru_aux(p['gru0_wih'], p['gru0_whh'], p['gru0_bih'], p['gru0_bhh'])
    gate_aux += [gru0[kk] for kk in _GRU_KEYS]
    xlo, xhi, hs, a_s, a_d = _rowmap(
        _make_t_nodeupd(True, True),
        [ulo, uhi, dsum[:, None], h0], gate_aux + _atom_pre_aux(0),
        [(32,), (32,), (64,), (1,), (1,)])

    for l in range(n_layers_m1):
        xjh, ev = _sc_gather_e(hs, a_s[:, 0], a_d[:, 0], src, dst)
        slo, shi, e1 = _rowmap(_t_scale_edge,
                               [xjh, ev[:, None]], [],
                               [(32,), (32,), (1,)])
        ulo, uhi, dsum = _sc_scatter_add(slo, shi, dst, e1[:, 0], n)
        aux = [p['atom_bias'][l][None, :]]
        grul = _gru_aux(p['atom_gru_wih'][l], p['atom_gru_whh'][l],
                        p['atom_gru_bih'][l], p['atom_gru_bhh'][l])
        aux += [grul[kk] for kk in _GRU_KEYS]
        xc_prev = jnp.concatenate([xlo, xhi], axis=1)
        last = l == n_layers_m1 - 1
        if not last:
            xlo, xhi, hs, a_s, a_d = _rowmap(
                _make_t_nodeupd(False, True),
                [ulo, uhi, dsum[:, None], xc_prev],
                aux + _atom_pre_aux(l + 1),
                [(32,), (32,), (64,), (1,), (1,)])
        else:
            xlo, xhi = _rowmap(_make_t_nodeupd(False, False),
                               [ulo, uhi, dsum[:, None], xc_prev], aux,
                               [(32,), (32,)])

    batch_pad = _pad_rows(batch, value=b)
    ulo, uhi = _sc_scatter_add(_pad_rows(xlo), _pad_rows(xhi),
                               batch_pad, None, b)
    out, t2 = _rowmap(_t_read_t2, [ulo, uhi],
                      [p['mol_lin_w'].T, p['mol_att_dst'][:, None]],
                      [(64,), (1,)], blk_target=1024)

    hlo, hhi, t1 = _rowmap(
        _t_ts_hoist, [xlo, xhi],
        [p['mol_lin_w'].T, p['mol_att_src'][:, None]],
        [(32,), (32,), (1,)])
    hlo_p, hhi_p, t1_p = (_pad_rows(hlo), _pad_rows(hhi),
                          _pad_rows(t1[:, 0]))

    molgru = _gru_aux(p['molgru_wih'], p['molgru_whh'],
                      p['molgru_bih'], p['molgru_bhh'])
    ts_post_aux = ([p['mol_bias'][None, :]]
                   + [molgru[kk] for kk in _GRU_KEYS]
                   + [p['mol_lin_w'].T, p['mol_att_dst'][:, None]])
    for _ in range(3):
        ulo, uhi, dsum = _sc_ts_fused(hlo_p, hhi_p, t1_p, t2[:, 0],
                                      batch_pad, b)
        out, t2 = _rowmap(_t_ts_post_t2,
                          [ulo, uhi, dsum[:, None], out], ts_post_aux,
                          [(64,), (1,)], blk_target=1024)

    logits, pct = _rowmap(
        _t_final, [out, f_mlp, d_mlp],
        [p['lin2_w'].T, p['lin2_b'][None, :], p['fus_w'].T,
         p['fus_b'][None, :], p['tox_w'].T, p['tox_b'][None, :],
         p['reg_w'].T, p['reg_b'][None, :]],
        [(13,), (1,)], blk_target=1024)
    return logits, pct[:, 0]


# speedup vs baseline: 1.6191x; 1.5770x over previous
import functools
import math

import jax
import jax.numpy as jnp
from jax import lax
from jax.experimental import pallas as pl
from jax.experimental.pallas import tpu as pltpu
from jax.experimental.pallas import tpu_sc as plsc

F32 = jnp.float32
CHUNK = 128
SC_NC = 2
SC_NS = 16
NW = SC_NC * SC_NS


def _pick_blk(m, target):
    best = None
    for d in range(1, int(math.isqrt(m)) + 1):
        if m % d == 0:
            for c in (d, m // d):
                if c <= target and c % 8 == 0 and (best is None or c > best):
                    best = c
    return best if best is not None else m


def _rowmap(body, row_ins, aux_ins, out_minors, blk_target=8000):
    m = row_ins[0].shape[0]
    blk = _pick_blk(m, blk_target)
    grid = (m // blk,)

    def _rspec(a):
        nd = a.ndim
        return pl.BlockSpec((blk,) + a.shape[1:],
                            lambda i, _nd=nd: (i,) + (0,) * (_nd - 1))

    def _aspec(a):
        nd = a.ndim
        return pl.BlockSpec(a.shape, lambda i, _nd=nd: (0,) * _nd)

    in_specs = [_rspec(a) for a in row_ins] + [_aspec(a) for a in aux_ins]
    out_shape = [jax.ShapeDtypeStruct((m,) + mi, F32) for mi in out_minors]
    out_specs = [pl.BlockSpec((blk,) + mi,
                              lambda i, _nd=len(mi): (i,) + (0,) * _nd)
                 for mi in out_minors]
    outs = pl.pallas_call(
        body, grid=grid, in_specs=in_specs, out_specs=out_specs,
        out_shape=out_shape,
    )(*row_ins, *aux_ins)
    return outs


def _lrelu(x):
    return jnp.maximum(x, 0.01 * x)


def _elu(x):
    return jnp.where(x > 0, x, jnp.exp(jnp.minimum(x, 0.0)) - 1.0)


def _gru_tc(x, h, w):
    i_r = x @ w['ihr'] + w['bihr']
    i_z = x @ w['ihz'] + w['bihz']
    i_n = x @ w['ihn'] + w['bihn']
    h_r = h @ w['hhr'] + w['bhhr']
    h_z = h @ w['hhz'] + w['bhhz']
    h_n = h @ w['hhn'] + w['bhhn']
    r = jax.nn.sigmoid(i_r + h_r)
    z = jax.nn.sigmoid(i_z + h_z)
    n = jnp.tanh(i_n + r * h_n)
    return (1.0 - z) * n + z * h


def _gru_aux(wih, whh, bih, bhh):
    H = wih.shape[1]
    return {
        'ihr': wih[0:H].T, 'ihz': wih[H:2 * H].T, 'ihn': wih[2 * H:].T,
        'hhr': whh[0:H].T, 'hhz': whh[H:2 * H].T, 'hhn': whh[2 * H:].T,
        'bihr': bih[None, 0:H], 'bihz': bih[None, H:2 * H],
        'bihn': bih[None, 2 * H:],
        'bhhr': bhh[None, 0:H], 'bhhz': bhh[None, H:2 * H],
        'bhhn': bhh[None, 2 * H:],
    }


_GRU_KEYS = ('ihr', 'ihz', 'ihn', 'hhr', 'hhz', 'hhn',
             'bihr', 'bihz', 'bihn', 'bhhr', 'bhhz', 'bhhn')


SB = 12
SBR = SB * CHUNK


def _sc_cp():
    return pltpu.CompilerParams(use_tc_tiling_on_sc=False)


def _sc_mesh():
    return plsc.VectorSubcoreMesh(core_axis_name="c", subcore_axis_name="s",
                                  num_cores=SC_NC, num_subcores=SC_NS)


def _rr(nsb, nworkers, wid, fn):
    nfull = nsb // nworkers
    rem = nsb % nworkers
    lax.fori_loop(0, nfull, lambda t, z: (fn(t * nworkers + wid), z)[1], 0)
    if rem:
        @pl.when(wid < rem)
        def _():
            fn(nfull * nworkers + wid)


def _exp_lrelu_block(av, bv, ev, nr):
    for q in range(nr // 16):
        sl = pl.ds(q * 16, 16)
        v = av[sl] + bv[sl]
        ev[sl] = jnp.exp(jnp.maximum(v, 0.01 * v))


def _sc_gather_pair(tab, rtab, src, dst):
    m = src.shape[0]
    nch = m // CHUNK
    nsb, remch = nch // SB, nch % SB
    d = tab.shape[1]
    src2 = src.reshape(nch, CHUNK)
    dst2 = dst.reshape(nch, CHUNK)

    @functools.partial(
        pl.kernel, mesh=_sc_mesh(), compiler_params=_sc_cp(),
        out_type=(jax.ShapeDtypeStruct((m, d), F32),
                  jax.ShapeDtypeStruct((m,), F32)),
        scratch_types=[pltpu.VMEM((SB, CHUNK), jnp.int32),
                       pltpu.VMEM((SB, CHUNK), jnp.int32),
                       pltpu.VMEM((SBR, d), F32),
                       pltpu.VMEM((SBR,), F32),
                       pltpu.SemaphoreType.DMA],
    )
    def k(tab_h, rtab_h, src_h, dst_h, rows_o, rg_o,
          sidx_v, didx_v, rows_v, rv_v, sem):
        wid = lax.axis_index("s") * SC_NC + lax.axis_index("c")

        def sblock(ch0, nj):
            nr = nj * CHUNK
            base = ch0 * CHUNK
            pltpu.sync_copy(src_h.at[pl.ds(ch0, nj)], sidx_v.at[pl.ds(0, nj)])
            pltpu.sync_copy(dst_h.at[pl.ds(ch0, nj)], didx_v.at[pl.ds(0, nj)])
            cps = [pltpu.async_copy(tab_h.at[sidx_v.at[j]],
                                    rows_v.at[pl.ds(j * CHUNK, CHUNK)], sem)
                   for j in range(nj)]
            cps += [pltpu.async_copy(rtab_h.at[didx_v.at[j]],
                                     rv_v.at[pl.ds(j * CHUNK, CHUNK)], sem)
                    for j in range(nj)]
            for cp in cps:
                cp.wait()
            pltpu.sync_copy(rows_v.at[pl.ds(0, nr)],
                            rows_o.at[pl.ds(base, nr)])
            pltpu.sync_copy(rv_v.at[pl.ds(0, nr)], rg_o.at[pl.ds(base, nr)])

        _rr(nsb, NW, wid, lambda sb: sblock(sb * SB, SB))
        if remch:
            @pl.when(wid == NW - 1)
            def _():
                sblock(nsb * SB, remch)

    return k(tab, rtab, src2, dst2)


def _sc_gather_e(tab, atab, btab, src, dst):
    m = src.shape[0]
    nch = m // CHUNK
    nsb, remch = nch // SB, nch % SB
    d = tab.shape[1]
    src2 = src.reshape(nch, CHUNK)
    dst2 = dst.reshape(nch, CHUNK)

    @functools.partial(
        pl.kernel, mesh=_sc_mesh(), compiler_params=_sc_cp(),
        out_type=(jax.ShapeDtypeStruct((m, d), F32),
                  jax.ShapeDtypeStruct((m,), F32)),
        scratch_types=[pltpu.VMEM((SB, CHUNK), jnp.int32),
                       pltpu.VMEM((SB, CHUNK), jnp.int32),
                       pltpu.VMEM((SBR, d), F32),
                       pltpu.VMEM((SBR,), F32),
                       pltpu.VMEM((SBR,), F32),
                       pltpu.VMEM((SBR,), F32),
                       pltpu.SemaphoreType.DMA],
    )
    def k(tab_h, atab_h, btab_h, src_h, dst_h, rows_o, e_o,
          sidx_v, didx_v, rows_v, av, bv, ev, sem):
        wid = lax.axis_index("s") * SC_NC + lax.axis_index("c")

        def sblock(ch0, nj):
            nr = nj * CHUNK
            base = ch0 * CHUNK
            pltpu.sync_copy(src_h.at[pl.ds(ch0, nj)], sidx_v.at[pl.ds(0, nj)])
            pltpu.sync_copy(dst_h.at[pl.ds(ch0, nj)], didx_v.at[pl.ds(0, nj)])
            cps = [pltpu.async_copy(tab_h.at[sidx_v.at[j]],
                                    rows_v.at[pl.ds(j * CHUNK, CHUNK)], sem)
                   for j in range(nj)]
            cps += [pltpu.async_copy(atab_h.at[sidx_v.at[j]],
                                     av.at[pl.ds(j * CHUNK, CHUNK)], sem)
                    for j in range(nj)]
            cps += [pltpu.async_copy(btab_h.at[didx_v.at[j]],
                                     bv.at[pl.ds(j * CHUNK, CHUNK)], sem)
                    for j in range(nj)]
            for cp in cps:
                cp.wait()
            _exp_lrelu_block(av, bv, ev, nr)
            pltpu.sync_copy(rows_v.at[pl.ds(0, nr)],
                            rows_o.at[pl.ds(base, nr)])
            pltpu.sync_copy(ev.at[pl.ds(0, nr)], e_o.at[pl.ds(base, nr)])

        _rr(nsb, NW, wid, lambda sb: sblock(sb * SB, SB))
        if remch:
            @pl.when(wid == NW - 1)
            def _():
                sblock(nsb * SB, remch)

    return k(tab, atab, btab, src2, dst2)


def _sc_layer(hlo, hhi, atab, btab, src, dst, nrows):
    m = src.shape[0]
    nch = m // CHUNK
    sbs = 4
    nsb, remch = nch // sbs, nch % sbs
    ra = nrows + 1
    src2 = src.reshape(nch, CHUNK)
    dst2 = dst.reshape(nch, CHUNK)

    zero_u = jnp.zeros((ra, 32), F32)
    zero_d = jnp.zeros((ra,), F32)

    @functools.partial(
        pl.kernel, mesh=_sc_mesh(), compiler_params=_sc_cp(),
        out_type=(jax.ShapeDtypeStruct((nrows, 32), F32),
                  jax.ShapeDtypeStruct((nrows, 32), F32),
                  jax.ShapeDtypeStruct((nrows,), F32)),
        scratch_types=[pltpu.VMEM_SHARED((ra, 32), F32),
                       pltpu.VMEM_SHARED((ra,), F32),
                       pltpu.VMEM((sbs, CHUNK), jnp.int32),
                       pltpu.VMEM((sbs, CHUNK), jnp.int32),
                       pltpu.VMEM((sbs * CHUNK, 32), F32),
                       pltpu.VMEM((sbs * CHUNK,), F32),
                       pltpu.VMEM((sbs * CHUNK,), F32),
                       pltpu.VMEM((sbs * CHUNK + 16,), F32),
                       pltpu.SemaphoreType.DMA],
    )
    def k(hlo_h, hhi_h, atab_h, btab_h, src_h, dst_h, zu_h, zd_h,
          ulo_h, uhi_h, d_h,
          acc_sh, dacc_sh, sidx_v, didx_v, rows_v, av, bv, ev, sem):
        cid = lax.axis_index("c")
        sid = lax.axis_index("s")

        @pl.when(sid == 0)
        def _():
            pltpu.sync_copy(zu_h, acc_sh)

        @pl.when(jnp.logical_and(sid == 0, cid == 0))
        def _():
            pltpu.sync_copy(zd_h, dacc_sh)
        plsc.subcore_barrier()

        def sblock(ch0, nj, tab_h, inc_d):
            nr = nj * CHUNK
            pltpu.sync_copy(src_h.at[pl.ds(ch0, nj)], sidx_v.at[pl.ds(0, nj)])
            pltpu.sync_copy(dst_h.at[pl.ds(ch0, nj)], didx_v.at[pl.ds(0, nj)])
            cps = [pltpu.async_copy(tab_h.at[sidx_v.at[j]],
                                    rows_v.at[pl.ds(j * CHUNK, CHUNK)], sem)
                   for j in range(nj)]
            cps += [pltpu.async_copy(atab_h.at[sidx_v.at[j]],
                                     av.at[pl.ds(j * CHUNK, CHUNK)], sem)
                    for j in range(nj)]
            cps += [pltpu.async_copy(btab_h.at[didx_v.at[j]],
                                     bv.at[pl.ds(j * CHUNK, CHUNK)], sem)
                    for j in range(nj)]
            for cp in cps:
                cp.wait()
            _exp_lrelu_block(av, bv, ev, nr)

            def scale(i, z):
                sc = ev[pl.ds(i, 16)][0]
                rows_v[i, pl.ds(0, 16)] = rows_v[i, pl.ds(0, 16)] * sc
                rows_v[i, pl.ds(16, 16)] = rows_v[i, pl.ds(16, 16)] * sc
                return z
            lax.fori_loop(0, nr, scale, 0)
            cps = [pltpu.async_copy(rows_v.at[pl.ds(j * CHUNK, CHUNK)],
                                    acc_sh.at[didx_v.at[j]], sem, add=True)
                   for j in range(nj)]
            if inc_d:
                cps += [pltpu.async_copy(ev.at[pl.ds(j * CHUNK, CHUNK)],
                                         dacc_sh.at[didx_v.at[j]], sem,
                                         add=True)
                        for j in range(nj)]
            for cp in cps:
                cp.wait()

        def run(tab_h, inc_d):
            _rr(nsb, SC_NS, sid,
                lambda sb: sblock(sb * sbs, sbs, tab_h, inc_d))
            if remch:
                @pl.when(sid == SC_NS - 1)
                def _():
                    sblock(nsb * sbs, remch, tab_h, inc_d)

        @pl.when(cid == 0)
        def _():
            run(hlo_h, True)

        @pl.when(cid == 1)
        def _():
            run(hhi_h, False)

        plsc.subcore_barrier()

        @pl.when(jnp.logical_and(sid == 0, cid == 0))
        def _():
            pltpu.sync_copy(acc_sh.at[pl.ds(0, nrows)], ulo_h)
            pltpu.sync_copy(dacc_sh.at[pl.ds(0, nrows)], d_h)

        @pl.when(jnp.logical_and(sid == 0, cid == 1))
        def _():
            pltpu.sync_copy(acc_sh.at[pl.ds(0, nrows)], uhi_h)

    return k(hlo, hhi, atab, btab, src2, dst2, zero_u, zero_d)


def _sc_scatter_add(vlo, vhi, idx, ew, nrows):
    m = idx.shape[0]
    nch = m // CHUNK
    sbs = 6
    nsb, remch = nch // sbs, nch % sbs
    ra = nrows + 1
    with_d = ew is not None
    idx2 = idx.reshape(nch, CHUNK)

    out_type = [jax.ShapeDtypeStruct((nrows, 32), F32),
                jax.ShapeDtypeStruct((nrows, 32), F32)]
    scratch = [pltpu.VMEM_SHARED((ra, 32), F32),
               pltpu.VMEM((sbs, CHUNK), jnp.int32),
               pltpu.VMEM((sbs * CHUNK, 32), F32),
               pltpu.SemaphoreType.DMA]
    if with_d:
        out_type.append(jax.ShapeDtypeStruct((nrows,), F32))
        scratch += [pltpu.VMEM((sbs * CHUNK,), F32),
                    pltpu.VMEM_SHARED((ra,), F32)]

    zero_u = jnp.zeros((ra, 32), F32)
    zero_d = jnp.zeros((ra,), F32)

    def body(refs):
        if with_d:
            (vlo_h, vhi_h, idx_h, ew_h, zu_h, zd_h, ulo_h, uhi_h, d_h,
             acc_sh, idxb_v, rows_v, sem, ew_v, dacc_sh) = refs
        else:
            (vlo_h, vhi_h, idx_h, zu_h, ulo_h, uhi_h,
             acc_sh, idxb_v, rows_v, sem) = refs
        cid = lax.axis_index("c")
        sid = lax.axis_index("s")

        @pl.when(sid == 0)
        def _():
            pltpu.sync_copy(zu_h, acc_sh)
        if with_d:
            @pl.when(jnp.logical_and(sid == 0, cid == 0))
            def _():
                pltpu.sync_copy(zd_h, dacc_sh)
        plsc.subcore_barrier()

        def sblock(ch0, nj, v_h, inc_d):
            nr = nj * CHUNK
            base = ch0 * CHUNK
            pltpu.sync_copy(idx_h.at[pl.ds(ch0, nj)], idxb_v.at[pl.ds(0, nj)])
            pltpu.sync_copy(v_h.at[pl.ds(base, nr)], rows_v.at[pl.ds(0, nr)])
            if inc_d:
                pltpu.sync_copy(ew_h.at[pl.ds(base, nr)], ew_v.at[pl.ds(0, nr)])
            cps = [pltpu.async_copy(rows_v.at[pl.ds(j * CHUNK, CHUNK)],
                                    acc_sh.at[idxb_v.at[j]], sem, add=True)
                   for j in range(nj)]
            if inc_d:
                cps += [pltpu.async_copy(ew_v.at[pl.ds(j * CHUNK, CHUNK)],
                                         dacc_sh.at[idxb_v.at[j]], sem,
                                         add=True)
                        for j in range(nj)]
            for cp in cps:
                cp.wait()

        def run(v_h, inc_d):
            _rr(nsb, SC_NS, sid,
                lambda sb: sblock(sb * sbs, sbs, v_h, inc_d))
            if remch:
                @pl.when(sid == SC_NS - 1)
                def _():
                    sblock(nsb * sbs, remch, v_h, inc_d)

        @pl.when(cid == 0)
        def _():
            run(vlo_h, with_d)

        @pl.when(cid == 1)
        def _():
            run(vhi_h, False)

        plsc.subcore_barrier()

        @pl.when(jnp.logical_and(sid == 0, cid == 0))
        def _():
            pltpu.sync_copy(acc_sh.at[pl.ds(0, nrows)], ulo_h)

        @pl.when(jnp.logical_and(sid == 0, cid == 1))
        def _():
            pltpu.sync_copy(acc_sh.at[pl.ds(0, nrows)], uhi_h)
        if with_d:
            @pl.when(jnp.logical_and(sid == 0, cid == 0))
            def _():
                pltpu.sync_copy(dacc_sh.at[pl.ds(0, nrows)], d_h)

    if with_d:
        @functools.partial(pl.kernel, mesh=_sc_mesh(), compiler_params=_sc_cp(),
                           out_type=tuple(out_type), scratch_types=scratch)
        def k(*refs):
            body(refs)
        return k(vlo, vhi, idx2, ew, zero_u, zero_d)
    else:
        @functools.partial(pl.kernel, mesh=_sc_mesh(), compiler_params=_sc_cp(),
                           out_type=tuple(out_type), scratch_types=scratch)
        def k(*refs):
            body(refs)
        return k(vlo, vhi, idx2, zero_u)


def _sc_ts_fused(hlo, hhi, t1, t2, bidx, nrows):
    m = bidx.shape[0]
    nch = m // CHUNK
    sbs = 8
    nsb, remch = nch // sbs, nch % sbs
    ra = nrows + 1
    idx2 = bidx.reshape(nch, CHUNK)

    zero_u = jnp.zeros((ra, 32), F32)
    zero_d = jnp.zeros((ra,), F32)

    @functools.partial(
        pl.kernel, mesh=_sc_mesh(), compiler_params=_sc_cp(),
        out_type=(jax.ShapeDtypeStruct((nrows, 32), F32),
                  jax.ShapeDtypeStruct((nrows, 32), F32),
                  jax.ShapeDtypeStruct((nrows,), F32)),
        scratch_types=[pltpu.VMEM_SHARED((ra, 32), F32),
                       pltpu.VMEM_SHARED((ra,), F32),
                       pltpu.VMEM((sbs, CHUNK), jnp.int32),
                       pltpu.VMEM((sbs * CHUNK, 32), F32),
                       pltpu.VMEM((sbs * CHUNK,), F32),
                       pltpu.VMEM((sbs * CHUNK,), F32),
                       pltpu.VMEM((sbs * CHUNK + 16,), F32),
                       pltpu.SemaphoreType.DMA],
    )
    def k(hlo_h, hhi_h, t1_h, t2_h, idx_h, zu_h, zd_h, ulo_h, uhi_h, d_h,
          acc_sh, dacc_sh, idxb_v, rows_v, av, bv, ev, sem):
        cid = lax.axis_index("c")
        sid = lax.axis_index("s")

        @pl.when(sid == 0)
        def _():
            pltpu.sync_copy(zu_h, acc_sh)

        @pl.when(jnp.logical_and(sid == 0, cid == 0))
        def _():
            pltpu.sync_copy(zd_h, dacc_sh)
        plsc.subcore_barrier()

        def sblock(ch0, nj, v_h, inc_d):
            nr = nj * CHUNK
            base = ch0 * CHUNK
            pltpu.sync_copy(idx_h.at[pl.ds(ch0, nj)], idxb_v.at[pl.ds(0, nj)])
            pltpu.sync_copy(v_h.at[pl.ds(base, nr)], rows_v.at[pl.ds(0, nr)])
            pltpu.sync_copy(t1_h.at[pl.ds(base, nr)], av.at[pl.ds(0, nr)])
            cps = [pltpu.async_copy(t2_h.at[idxb_v.at[j]],
                                    bv.at[pl.ds(j * CHUNK, CHUNK)], sem)
                   for j in range(nj)]
            for cp in cps:
                cp.wait()
            _exp_lrelu_block(av, bv, ev, nr)

            def scale(i, z):
                s = ev[pl.ds(i, 16)][0]
                rows_v[i, pl.ds(0, 16)] = rows_v[i, pl.ds(0, 16)] * s
                rows_v[i, pl.ds(16, 16)] = rows_v[i, pl.ds(16, 16)] * s
                return z
            lax.fori_loop(0, nr, scale, 0)
            cps = [pltpu.async_copy(rows_v.at[pl.ds(j * CHUNK, CHUNK)],
                                    acc_sh.at[idxb_v.at[j]], sem, add=True)
                   for j in range(nj)]
            if inc_d:
                cps += [pltpu.async_copy(ev.at[pl.ds(j * CHUNK, CHUNK)],
                                         dacc_sh.at[idxb_v.at[j]], sem,
                                         add=True)
                        for j in range(nj)]
            for cp in cps:
                cp.wait()

        def run(v_h, inc_d):
            _rr(nsb, SC_NS, sid,
                lambda sb: sblock(sb * sbs, sbs, v_h, inc_d))
            if remch:
                @pl.when(sid == SC_NS - 1)
                def _():
                    sblock(nsb * sbs, remch, v_h, inc_d)

        @pl.when(cid == 0)
        def _():
            run(hlo_h, True)

        @pl.when(cid == 1)
        def _():
            run(hhi_h, False)

        plsc.subcore_barrier()

        @pl.when(jnp.logical_and(sid == 0, cid == 0))
        def _():
            pltpu.sync_copy(acc_sh.at[pl.ds(0, nrows)], ulo_h)
            pltpu.sync_copy(dacc_sh.at[pl.ds(0, nrows)], d_h)

        @pl.when(jnp.logical_and(sid == 0, cid == 1))
        def _():
            pltpu.sync_copy(acc_sh.at[pl.ds(0, nrows)], uhi_h)

    return k(hlo, hhi, t1, t2, idx2, zero_u, zero_d)


def _pad_rows(a, mult=CHUNK, value=0):
    m = a.shape[0]
    pad = (-m) % mult
    if pad == 0:
        return a
    cfg = [(0, pad)] + [(0, 0)] * (a.ndim - 1)
    return jnp.pad(a, cfg, constant_values=value)


def _t_h0(x_r, lin1T, lin1b, attr, h0_r, r_r):
    h0 = _lrelu(x_r[...] @ lin1T[...] + lin1b[...])
    h0_r[...] = h0
    r_r[...] = h0 @ attr[...]


def _t_gate(xj_r, ea_r, rg_r, w1aT, w1bT, attl, slo_r, shi_r, e1_r):
    m = _lrelu(xj_r[...] @ w1aT[...] + ea_r[...] @ w1bT[...])
    ma = m @ attl[...]
    e = jnp.exp(_lrelu(ma + rg_r[...]))
    s = e * m
    slo_r[...] = s[:, :32]
    shi_r[...] = s[:, 32:]
    e1_r[...] = e


def _t_scale_edge(xjh_r, e_r, slo_r, shi_r, e1_r):
    e = e_r[...]
    s = e * xjh_r[...]
    slo_r[...] = s[:, :32]
    shi_r[...] = s[:, 32:]
    e1_r[...] = e


def _make_t_nodeupd(project, nxt):
    def body(*refs):
        (ulo_r, uhi_r, d_r, h_r) = refs[:4]
        i = 4
        if project:
            lin2T, bias = refs[i], refs[i + 1]
            i += 2
        else:
            bias = refs[i]
            i += 1
        gru = dict(zip(_GRU_KEYS, refs[i:i + 12]))
        i += 12
        if nxt:
            wT, asrc, adst = refs[i], refs[i + 1], refs[i + 2]
            i += 3
        outs = refs[i:]
        u = jnp.concatenate([ulo_r[...], uhi_r[...]], axis=1)
        agg = u / (d_r[...] + 1e-16)
        if project:
            agg = agg @ lin2T[...] + bias[...]
        else:
            agg = agg + bias[...]
        hcand = _elu(agg)
        gw = {kk: vv[...] for kk, vv in gru.items()}
        xc = jnp.maximum(_gru_tc(hcand, h_r[...], gw), 0.0)
        outs[0][...] = xc[:, :32]
        outs[1][...] = xc[:, 32:]
        if nxt:
            hs = xc @ wT[...]
            outs[2][...] = hs[:, :32]
            outs[3][...] = hs[:, 32:]
            outs[4][...] = hs @ asrc[...]
            outs[5][...] = hs @ adst[...]
    return body


def _t_ts_hoist(xlo_r, xhi_r, molT, asrc, hlo_r, hhi_r, t1_r):
    xc = jnp.concatenate([xlo_r[...], xhi_r[...]], axis=1)
    hs = xc @ molT[...]
    hlo_r[...] = hs[:, :32]
    hhi_r[...] = hs[:, 32:]
    t1_r[...] = hs @ asrc[...]


def _t_read_t2(ulo_r, uhi_r, molT, adst, out_r, t2_r):
    out = jnp.maximum(
        jnp.concatenate([ulo_r[...], uhi_r[...]], axis=1), 0.0)
    out_r[...] = out
    t2_r[...] = (out @ molT[...]) @ adst[...]


def _t_ts_post_t2(*refs):
    ulo_r, uhi_r, d_r, out_r = refs[:4]
    bias = refs[4]
    gru = dict(zip(_GRU_KEYS, refs[5:17]))
    molT, adst = refs[17], refs[18]
    newout_r, t2_r = refs[19], refs[20]
    u = jnp.concatenate([ulo_r[...], uhi_r[...]], axis=1)
    h = _elu(u / (d_r[...] + 1e-16) + bias[...])
    gw = {kk: vv[...] for kk, vv in gru.items()}
    out = jnp.maximum(_gru_tc(h, out_r[...], gw), 0.0)
    newout_r[...] = out
    t2_r[...] = (out @ molT[...]) @ adst[...]


_BNF = 1.0 / math.sqrt(1.0 + 1e-5)


def _t_fd(fp_r, de_r, fw1T, fb1, fg, fbb, fw2T, fb2,
          dw1T, db1, dg, dbb, dw2T, db2, f_r, d_r):
    f = jnp.maximum(fp_r[...] @ fw1T[...] + fb1[...], 0.0)
    f = f * _BNF * fg[...] + fbb[...]
    f_r[...] = jnp.maximum(f @ fw2T[...] + fb2[...], 0.0)
    d = jnp.maximum(de_r[...] @ dw1T[...] + db1[...], 0.0)
    d = d * _BNF * dg[...] + dbb[...]
    d_r[...] = jnp.maximum(d @ dw2T[...] + db2[...], 0.0)


def _t_final(out_r, f_r, d_r, lin2T, lin2b, fusT, fusb,
             toxT, toxb, regT, regb, logits_r, pct_r):
    ge = out_r[...] @ lin2T[...] + lin2b[...]
    comb = jnp.concatenate([ge, f_r[...], d_r[...]], axis=1)
    shared = jnp.maximum(comb @ fusT[...] + fusb[...], 0.0)
    logits_r[...] = shared @ toxT[...] + toxb[...]
    pct_r[...] = shared @ regT[...] + regb[...]


def kernel(x, edge_index, edge_attr, batch, fingerprints, descriptors, params):
    p = params
    n = x.shape[0]
    b = fingerprints.shape[0]
    n_layers_m1 = p['atom_lin_w'].shape[0]

    src = edge_index[0].astype(jnp.int32)
    dst = edge_index[1].astype(jnp.int32)
    batch = batch.astype(jnp.int32)

    f_mlp, d_mlp = _rowmap(
        _t_fd, [fingerprints, descriptors],
        [p['fp_w1'].T, p['fp_b1'][None, :], p['fp_bn_g'][None, :],
         p['fp_bn_b'][None, :], p['fp_w2'].T, p['fp_b2'][None, :],
         p['desc_w1'].T, p['desc_b1'][None, :], p['desc_bn_g'][None, :],
         p['desc_bn_b'][None, :], p['desc_w2'].T, p['desc_b2'][None, :]],
        [(64,), (64,)], blk_target=1024)

    h0, r = _rowmap(_t_h0, [x],
                    [p['lin1_w'].T, p['lin1_b'][None, :],
                     p['gate_att_r'][:, None]],
                    [(64,), (1,)])

    xj, rg = _sc_gather_pair(h0, r[:, 0], src, dst)
    slo, shi, e1 = _rowmap(
        _t_gate, [xj, edge_attr, rg[:, None]],
        [p['gate_lin1_w'][:, :64].T, p['gate_lin1_w'][:, 64:].T,
         p['gate_att_l'][:, None]],
        [(32,), (32,), (1,)])
    ulo, uhi, dsum = _sc_scatter_add(slo, shi, dst, e1[:, 0], n)

    def _atom_pre_aux(l):
        return [p['atom_lin_w'][l].T, p['atom_att_src'][l][:, None],
                p['atom_att_dst'][l][:, None]]

    gate_aux = [p['gate_lin2_w'].T, p['gate_bias'][None, :]]
    gru0 = _gru_aux(p['gru0_wih'], p['gru0_whh'], p['gru0_bih'], p['gru0_bhh'])
    gate_aux += [gru0[kk] for kk in _GRU_KEYS]
    xlo, xhi, hs_lo, hs_hi, a_s, a_d = _rowmap(
        _make_t_nodeupd(True, True),
        [ulo, uhi, dsum[:, None], h0], gate_aux + _atom_pre_aux(0),
        [(32,), (32,), (32,), (32,), (1,), (1,)])

    for l in range(n_layers_m1):
        ulo, uhi, dsum = _sc_layer(hs_lo, hs_hi, a_s[:, 0], a_d[:, 0],
                                   src, dst, n)
        aux = [p['atom_bias'][l][None, :]]
        grul = _gru_aux(p['atom_gru_wih'][l], p['atom_gru_whh'][l],
                        p['atom_gru_bih'][l], p['atom_gru_bhh'][l])
        aux += [grul[kk] for kk in _GRU_KEYS]
        xc_prev = jnp.concatenate([xlo, xhi], axis=1)
        last = l == n_layers_m1 - 1
        if not last:
            xlo, xhi, hs_lo, hs_hi, a_s, a_d = _rowmap(
                _make_t_nodeupd(False, True),
                [ulo, uhi, dsum[:, None], xc_prev],
                aux + _atom_pre_aux(l + 1),
                [(32,), (32,), (32,), (32,), (1,), (1,)])
        else:
            xlo, xhi = _rowmap(_make_t_nodeupd(False, False),
                               [ulo, uhi, dsum[:, None], xc_prev], aux,
                               [(32,), (32,)])

    batch_pad = _pad_rows(batch, value=b)
    ulo, uhi = _sc_scatter_add(_pad_rows(xlo), _pad_rows(xhi),
                               batch_pad, None, b)
    out, t2 = _rowmap(_t_read_t2, [ulo, uhi],
                      [p['mol_lin_w'].T, p['mol_att_dst'][:, None]],
                      [(64,), (1,)], blk_target=1024)

    hlo, hhi, t1 = _rowmap(
        _t_ts_hoist, [xlo, xhi],
        [p['mol_lin_w'].T, p['mol_att_src'][:, None]],
        [(32,), (32,), (1,)])
    hlo_p, hhi_p, t1_p = (_pad_rows(hlo), _pad_rows(hhi),
                          _pad_rows(t1[:, 0]))

    molgru = _gru_aux(p['molgru_wih'], p['molgru_whh'],
                      p['molgru_bih'], p['molgru_bhh'])
    ts_post_aux = ([p['mol_bias'][None, :]]
                   + [molgru[kk] for kk in _GRU_KEYS]
                   + [p['mol_lin_w'].T, p['mol_att_dst'][:, None]])
    for _ in range(3):
        ulo, uhi, dsum = _sc_ts_fused(hlo_p, hhi_p, t1_p, t2[:, 0],
                                      batch_pad, b)
        out, t2 = _rowmap(_t_ts_post_t2,
                          [ulo, uhi, dsum[:, None], out], ts_post_aux,
                          [(64,), (1,)], blk_target=1024)

    logits, pct = _rowmap(
        _t_final, [out, f_mlp, d_mlp],
        [p['lin2_w'].T, p['lin2_b'][None, :], p['fus_w'].T,
         p['fus_b'][None, :], p['tox_w'].T, p['tox_b'][None, :],
         p['reg_w'].T, p['reg_b'][None, :]],
        [(13,), (1,)], blk_target=1024)
    return logits, pct[:, 0]


# speedup vs baseline: 1.6505x; 1.0193x over previous
import functools
import math

import jax
import jax.numpy as jnp
from jax import lax
from jax.experimental import pallas as pl
from jax.experimental.pallas import tpu as pltpu
from jax.experimental.pallas import tpu_sc as plsc

F32 = jnp.float32
CHUNK = 128
SC_NC = 2
SC_NS = 16
NW = SC_NC * SC_NS


def _pick_blk(m, target):
    best = None
    for d in range(1, int(math.isqrt(m)) + 1):
        if m % d == 0:
            for c in (d, m // d):
                if c <= target and c % 8 == 0 and (best is None or c > best):
                    best = c
    return best if best is not None else m


def _rowmap(body, row_ins, aux_ins, out_minors, blk_target=8000):
    m = row_ins[0].shape[0]
    blk = _pick_blk(m, blk_target)
    grid = (m // blk,)

    def _rspec(a):
        nd = a.ndim
        return pl.BlockSpec((blk,) + a.shape[1:],
                            lambda i, _nd=nd: (i,) + (0,) * (_nd - 1))

    def _aspec(a):
        nd = a.ndim
        return pl.BlockSpec(a.shape, lambda i, _nd=nd: (0,) * _nd)

    in_specs = [_rspec(a) for a in row_ins] + [_aspec(a) for a in aux_ins]
    out_shape = [jax.ShapeDtypeStruct((m,) + mi, F32) for mi in out_minors]
    out_specs = [pl.BlockSpec((blk,) + mi,
                              lambda i, _nd=len(mi): (i,) + (0,) * _nd)
                 for mi in out_minors]
    outs = pl.pallas_call(
        body, grid=grid, in_specs=in_specs, out_specs=out_specs,
        out_shape=out_shape,
    )(*row_ins, *aux_ins)
    return outs


def _lrelu(x):
    return jnp.maximum(x, 0.01 * x)


def _elu(x):
    return jnp.where(x > 0, x, jnp.exp(jnp.minimum(x, 0.0)) - 1.0)


def _gru_tc(x, h, w):
    i_r = x @ w['ihr'] + w['bihr']
    i_z = x @ w['ihz'] + w['bihz']
    i_n = x @ w['ihn'] + w['bihn']
    h_r = h @ w['hhr'] + w['bhhr']
    h_z = h @ w['hhz'] + w['bhhz']
    h_n = h @ w['hhn'] + w['bhhn']
    r = jax.nn.sigmoid(i_r + h_r)
    z = jax.nn.sigmoid(i_z + h_z)
    n = jnp.tanh(i_n + r * h_n)
    return (1.0 - z) * n + z * h


def _gru_aux(wih, whh, bih, bhh):
    H = wih.shape[1]
    return {
        'ihr': wih[0:H].T, 'ihz': wih[H:2 * H].T, 'ihn': wih[2 * H:].T,
        'hhr': whh[0:H].T, 'hhz': whh[H:2 * H].T, 'hhn': whh[2 * H:].T,
        'bihr': bih[None, 0:H], 'bihz': bih[None, H:2 * H],
        'bihn': bih[None, 2 * H:],
        'bhhr': bhh[None, 0:H], 'bhhz': bhh[None, H:2 * H],
        'bhhn': bhh[None, 2 * H:],
    }


_GRU_KEYS = ('ihr', 'ihz', 'ihn', 'hhr', 'hhz', 'hhn',
             'bihr', 'bihz', 'bihn', 'bhhr', 'bhhz', 'bhhn')


SB = 12
SBR = SB * CHUNK


def _sc_cp():
    return pltpu.CompilerParams(use_tc_tiling_on_sc=False)


def _sc_mesh():
    return plsc.VectorSubcoreMesh(core_axis_name="c", subcore_axis_name="s",
                                  num_cores=SC_NC, num_subcores=SC_NS)


def _rr(nsb, nworkers, wid, fn):
    nfull = nsb // nworkers
    rem = nsb % nworkers
    lax.fori_loop(0, nfull, lambda t, z: (fn(t * nworkers + wid), z)[1], 0)
    if rem:
        @pl.when(wid < rem)
        def _():
            fn(nfull * nworkers + wid)


def _exp_lrelu_block(av, bv, ev, nr):
    for q in range(nr // 16):
        sl = pl.ds(q * 16, 16)
        v = av[sl] + bv[sl]
        ev[sl] = jnp.exp(jnp.maximum(v, 0.01 * v))


def _sc_gather_pair(tab, rtab, src, dst):
    m = src.shape[0]
    nch = m // CHUNK
    nsb, remch = nch // SB, nch % SB
    d = tab.shape[1]
    src2 = src.reshape(nch, CHUNK)
    dst2 = dst.reshape(nch, CHUNK)

    @functools.partial(
        pl.kernel, mesh=_sc_mesh(), compiler_params=_sc_cp(),
        out_type=(jax.ShapeDtypeStruct((m, d), F32),
                  jax.ShapeDtypeStruct((m,), F32)),
        scratch_types=[pltpu.VMEM((SB, CHUNK), jnp.int32),
                       pltpu.VMEM((SB, CHUNK), jnp.int32),
                       pltpu.VMEM((SBR, d), F32),
                       pltpu.VMEM((SBR,), F32),
                       pltpu.SemaphoreType.DMA],
    )
    def k(tab_h, rtab_h, src_h, dst_h, rows_o, rg_o,
          sidx_v, didx_v, rows_v, rv_v, sem):
        wid = lax.axis_index("s") * SC_NC + lax.axis_index("c")

        def sblock(ch0, nj):
            nr = nj * CHUNK
            base = ch0 * CHUNK
            pltpu.sync_copy(src_h.at[pl.ds(ch0, nj)], sidx_v.at[pl.ds(0, nj)])
            pltpu.sync_copy(dst_h.at[pl.ds(ch0, nj)], didx_v.at[pl.ds(0, nj)])
            cps = [pltpu.async_copy(tab_h.at[sidx_v.at[j]],
                                    rows_v.at[pl.ds(j * CHUNK, CHUNK)], sem)
                   for j in range(nj)]
            cps += [pltpu.async_copy(rtab_h.at[didx_v.at[j]],
                                     rv_v.at[pl.ds(j * CHUNK, CHUNK)], sem)
                    for j in range(nj)]
            for cp in cps:
                cp.wait()
            pltpu.sync_copy(rows_v.at[pl.ds(0, nr)],
                            rows_o.at[pl.ds(base, nr)])
            pltpu.sync_copy(rv_v.at[pl.ds(0, nr)], rg_o.at[pl.ds(base, nr)])

        _rr(nsb, NW, wid, lambda sb: sblock(sb * SB, SB))
        if remch:
            @pl.when(wid == NW - 1)
            def _():
                sblock(nsb * SB, remch)

    return k(tab, rtab, src2, dst2)


def _sc_gather_e(tab, atab, btab, src, dst):
    m = src.shape[0]
    nch = m // CHUNK
    nsb, remch = nch // SB, nch % SB
    d = tab.shape[1]
    src2 = src.reshape(nch, CHUNK)
    dst2 = dst.reshape(nch, CHUNK)

    @functools.partial(
        pl.kernel, mesh=_sc_mesh(), compiler_params=_sc_cp(),
        out_type=(jax.ShapeDtypeStruct((m, d), F32),
                  jax.ShapeDtypeStruct((m,), F32)),
        scratch_types=[pltpu.VMEM((SB, CHUNK), jnp.int32),
                       pltpu.VMEM((SB, CHUNK), jnp.int32),
                       pltpu.VMEM((SBR, d), F32),
                       pltpu.VMEM((SBR,), F32),
                       pltpu.VMEM((SBR,), F32),
                       pltpu.VMEM((SBR,), F32),
                       pltpu.SemaphoreType.DMA],
    )
    def k(tab_h, atab_h, btab_h, src_h, dst_h, rows_o, e_o,
          sidx_v, didx_v, rows_v, av, bv, ev, sem):
        wid = lax.axis_index("s") * SC_NC + lax.axis_index("c")

        def sblock(ch0, nj):
            nr = nj * CHUNK
            base = ch0 * CHUNK
            pltpu.sync_copy(src_h.at[pl.ds(ch0, nj)], sidx_v.at[pl.ds(0, nj)])
            pltpu.sync_copy(dst_h.at[pl.ds(ch0, nj)], didx_v.at[pl.ds(0, nj)])
            cps = [pltpu.async_copy(tab_h.at[sidx_v.at[j]],
                                    rows_v.at[pl.ds(j * CHUNK, CHUNK)], sem)
                   for j in range(nj)]
            cps += [pltpu.async_copy(atab_h.at[sidx_v.at[j]],
                                     av.at[pl.ds(j * CHUNK, CHUNK)], sem)
                    for j in range(nj)]
            cps += [pltpu.async_copy(btab_h.at[didx_v.at[j]],
                                     bv.at[pl.ds(j * CHUNK, CHUNK)], sem)
                    for j in range(nj)]
            for cp in cps:
                cp.wait()
            _exp_lrelu_block(av, bv, ev, nr)
            pltpu.sync_copy(rows_v.at[pl.ds(0, nr)],
                            rows_o.at[pl.ds(base, nr)])
            pltpu.sync_copy(ev.at[pl.ds(0, nr)], e_o.at[pl.ds(base, nr)])

        _rr(nsb, NW, wid, lambda sb: sblock(sb * SB, SB))
        if remch:
            @pl.when(wid == NW - 1)
            def _():
                sblock(nsb * SB, remch)

    return k(tab, atab, btab, src2, dst2)


def _sc_layer(hlo, hhi, atab, btab, src, dst, nrows):
    m = src.shape[0]
    nch = m // CHUNK
    sbs = 5
    nsb, remch = nch // sbs, nch % sbs
    ra = nrows + 1
    src2 = src.reshape(nch, CHUNK)
    dst2 = dst.reshape(nch, CHUNK)

    zero_u = jnp.zeros((ra, 32), F32)
    zero_d = jnp.zeros((ra,), F32)

    @functools.partial(
        pl.kernel, mesh=_sc_mesh(), compiler_params=_sc_cp(),
        out_type=(jax.ShapeDtypeStruct((nrows, 32), F32),
                  jax.ShapeDtypeStruct((nrows, 32), F32),
                  jax.ShapeDtypeStruct((nrows,), F32)),
        scratch_types=[pltpu.VMEM_SHARED((ra, 32), F32),
                       pltpu.VMEM_SHARED((ra,), F32),
                       pltpu.VMEM((sbs, CHUNK), jnp.int32),
                       pltpu.VMEM((sbs, CHUNK), jnp.int32),
                       pltpu.VMEM((sbs * CHUNK, 32), F32),
                       pltpu.VMEM((sbs * CHUNK,), F32),
                       pltpu.VMEM((sbs * CHUNK,), F32),
                       pltpu.VMEM((sbs * CHUNK + 16,), F32),
                       pltpu.SemaphoreType.DMA],
    )
    def k(hlo_h, hhi_h, atab_h, btab_h, src_h, dst_h, zu_h, zd_h,
          ulo_h, uhi_h, d_h,
          acc_sh, dacc_sh, sidx_v, didx_v, rows_v, av, bv, ev, sem):
        cid = lax.axis_index("c")
        sid = lax.axis_index("s")

        @pl.when(sid == 0)
        def _():
            pltpu.sync_copy(zu_h, acc_sh)

        @pl.when(jnp.logical_and(sid == 0, cid == 0))
        def _():
            pltpu.sync_copy(zd_h, dacc_sh)
        plsc.subcore_barrier()

        def sblock(ch0, nj, tab_h, inc_d):
            nr = nj * CHUNK
            pltpu.sync_copy(src_h.at[pl.ds(ch0, nj)], sidx_v.at[pl.ds(0, nj)])
            pltpu.sync_copy(dst_h.at[pl.ds(ch0, nj)], didx_v.at[pl.ds(0, nj)])
            cps = [pltpu.async_copy(tab_h.at[sidx_v.at[j]],
                                    rows_v.at[pl.ds(j * CHUNK, CHUNK)], sem)
                   for j in range(nj)]
            cps += [pltpu.async_copy(atab_h.at[sidx_v.at[j]],
                                     av.at[pl.ds(j * CHUNK, CHUNK)], sem)
                    for j in range(nj)]
            cps += [pltpu.async_copy(btab_h.at[didx_v.at[j]],
                                     bv.at[pl.ds(j * CHUNK, CHUNK)], sem)
                    for j in range(nj)]
            for cp in cps:
                cp.wait()
            _exp_lrelu_block(av, bv, ev, nr)

            def scale8(i8, z):
                for q in range(8):
                    i = i8 * 8 + q
                    sc = ev[pl.ds(i, 16)][0]
                    rows_v[i, pl.ds(0, 16)] = rows_v[i, pl.ds(0, 16)] * sc
                    rows_v[i, pl.ds(16, 16)] = rows_v[i, pl.ds(16, 16)] * sc
                return z
            lax.fori_loop(0, nr // 8, scale8, 0)
            cps = [pltpu.async_copy(rows_v.at[pl.ds(j * CHUNK, CHUNK)],
                                    acc_sh.at[didx_v.at[j]], sem, add=True)
                   for j in range(nj)]
            if inc_d:
                cps += [pltpu.async_copy(ev.at[pl.ds(j * CHUNK, CHUNK)],
                                         dacc_sh.at[didx_v.at[j]], sem,
                                         add=True)
                        for j in range(nj)]
            for cp in cps:
                cp.wait()

        def run(tab_h, inc_d):
            _rr(nsb, SC_NS, sid,
                lambda sb: sblock(sb * sbs, sbs, tab_h, inc_d))
            if remch:
                @pl.when(sid == SC_NS - 1)
                def _():
                    sblock(nsb * sbs, remch, tab_h, inc_d)

        @pl.when(cid == 0)
        def _():
            run(hlo_h, True)

        @pl.when(cid == 1)
        def _():
            run(hhi_h, False)

        plsc.subcore_barrier()

        @pl.when(jnp.logical_and(sid == 0, cid == 0))
        def _():
            pltpu.sync_copy(acc_sh.at[pl.ds(0, nrows)], ulo_h)
            pltpu.sync_copy(dacc_sh.at[pl.ds(0, nrows)], d_h)

        @pl.when(jnp.logical_and(sid == 0, cid == 1))
        def _():
            pltpu.sync_copy(acc_sh.at[pl.ds(0, nrows)], uhi_h)

    return k(hlo, hhi, atab, btab, src2, dst2, zero_u, zero_d)


def _sc_scatter_add(vlo, vhi, idx, ew, nrows):
    m = idx.shape[0]
    nch = m // CHUNK
    sbs = 6
    nsb, remch = nch // sbs, nch % sbs
    ra = nrows + 1
    with_d = ew is not None
    idx2 = idx.reshape(nch, CHUNK)

    out_type = [jax.ShapeDtypeStruct((nrows, 32), F32),
                jax.ShapeDtypeStruct((nrows, 32), F32)]
    scratch = [pltpu.VMEM_SHARED((ra, 32), F32),
               pltpu.VMEM((sbs, CHUNK), jnp.int32),
               pltpu.VMEM((sbs * CHUNK, 32), F32),
               pltpu.SemaphoreType.DMA]
    if with_d:
        out_type.append(jax.ShapeDtypeStruct((nrows,), F32))
        scratch += [pltpu.VMEM((sbs * CHUNK,), F32),
                    pltpu.VMEM_SHARED((ra,), F32)]

    zero_u = jnp.zeros((ra, 32), F32)
    zero_d = jnp.zeros((ra,), F32)

    def body(refs):
        if with_d:
            (vlo_h, vhi_h, idx_h, ew_h, zu_h, zd_h, ulo_h, uhi_h, d_h,
             acc_sh, idxb_v, rows_v, sem, ew_v, dacc_sh) = refs
        else:
            (vlo_h, vhi_h, idx_h, zu_h, ulo_h, uhi_h,
             acc_sh, idxb_v, rows_v, sem) = refs
        cid = lax.axis_index("c")
        sid = lax.axis_index("s")

        @pl.when(sid == 0)
        def _():
            pltpu.sync_copy(zu_h, acc_sh)
        if with_d:
            @pl.when(jnp.logical_and(sid == 0, cid == 0))
            def _():
                pltpu.sync_copy(zd_h, dacc_sh)
        plsc.subcore_barrier()

        def sblock(ch0, nj, v_h, inc_d):
            nr = nj * CHUNK
            base = ch0 * CHUNK
            pltpu.sync_copy(idx_h.at[pl.ds(ch0, nj)], idxb_v.at[pl.ds(0, nj)])
            pltpu.sync_copy(v_h.at[pl.ds(base, nr)], rows_v.at[pl.ds(0, nr)])
            if inc_d:
                pltpu.sync_copy(ew_h.at[pl.ds(base, nr)], ew_v.at[pl.ds(0, nr)])
            cps = [pltpu.async_copy(rows_v.at[pl.ds(j * CHUNK, CHUNK)],
                                    acc_sh.at[idxb_v.at[j]], sem, add=True)
                   for j in range(nj)]
            if inc_d:
                cps += [pltpu.async_copy(ew_v.at[pl.ds(j * CHUNK, CHUNK)],
                                         dacc_sh.at[idxb_v.at[j]], sem,
                                         add=True)
                        for j in range(nj)]
            for cp in cps:
                cp.wait()

        def run(v_h, inc_d):
            _rr(nsb, SC_NS, sid,
                lambda sb: sblock(sb * sbs, sbs, v_h, inc_d))
            if remch:
                @pl.when(sid == SC_NS - 1)
                def _():
                    sblock(nsb * sbs, remch, v_h, inc_d)

        @pl.when(cid == 0)
        def _():
            run(vlo_h, with_d)

        @pl.when(cid == 1)
        def _():
            run(vhi_h, False)

        plsc.subcore_barrier()

        @pl.when(jnp.logical_and(sid == 0, cid == 0))
        def _():
            pltpu.sync_copy(acc_sh.at[pl.ds(0, nrows)], ulo_h)

        @pl.when(jnp.logical_and(sid == 0, cid == 1))
        def _():
            pltpu.sync_copy(acc_sh.at[pl.ds(0, nrows)], uhi_h)
        if with_d:
            @pl.when(jnp.logical_and(sid == 0, cid == 0))
            def _():
                pltpu.sync_copy(dacc_sh.at[pl.ds(0, nrows)], d_h)

    if with_d:
        @functools.partial(pl.kernel, mesh=_sc_mesh(), compiler_params=_sc_cp(),
                           out_type=tuple(out_type), scratch_types=scratch)
        def k(*refs):
            body(refs)
        return k(vlo, vhi, idx2, ew, zero_u, zero_d)
    else:
        @functools.partial(pl.kernel, mesh=_sc_mesh(), compiler_params=_sc_cp(),
                           out_type=tuple(out_type), scratch_types=scratch)
        def k(*refs):
            body(refs)
        return k(vlo, vhi, idx2, zero_u)


def _sc_ts_fused(hlo, hhi, t1, t2, bidx, nrows):
    m = bidx.shape[0]
    nch = m // CHUNK
    sbs = 8
    nsb, remch = nch // sbs, nch % sbs
    ra = nrows + 1
    idx2 = bidx.reshape(nch, CHUNK)

    zero_u = jnp.zeros((ra, 32), F32)
    zero_d = jnp.zeros((ra,), F32)

    @functools.partial(
        pl.kernel, mesh=_sc_mesh(), compiler_params=_sc_cp(),
        out_type=(jax.ShapeDtypeStruct((nrows, 32), F32),
                  jax.ShapeDtypeStruct((nrows, 32), F32),
                  jax.ShapeDtypeStruct((nrows,), F32)),
        scratch_types=[pltpu.VMEM_SHARED((ra, 32), F32),
                       pltpu.VMEM_SHARED((ra,), F32),
                       pltpu.VMEM((sbs, CHUNK), jnp.int32),
                       pltpu.VMEM((sbs * CHUNK, 32), F32),
                       pltpu.VMEM((sbs * CHUNK,), F32),
                       pltpu.VMEM((sbs * CHUNK,), F32),
                       pltpu.VMEM((sbs * CHUNK + 16,), F32),
                       pltpu.SemaphoreType.DMA],
    )
    def k(hlo_h, hhi_h, t1_h, t2_h, idx_h, zu_h, zd_h, ulo_h, uhi_h, d_h,
          acc_sh, dacc_sh, idxb_v, rows_v, av, bv, ev, sem):
        cid = lax.axis_index("c")
        sid = lax.axis_index("s")

        @pl.when(sid == 0)
        def _():
            pltpu.sync_copy(zu_h, acc_sh)

        @pl.when(jnp.logical_and(sid == 0, cid == 0))
        def _():
            pltpu.sync_copy(zd_h, dacc_sh)
        plsc.subcore_barrier()

        def sblock(ch0, nj, v_h, inc_d):
            nr = nj * CHUNK
            base = ch0 * CHUNK
            pltpu.sync_copy(idx_h.at[pl.ds(ch0, nj)], idxb_v.at[pl.ds(0, nj)])
            pltpu.sync_copy(v_h.at[pl.ds(base, nr)], rows_v.at[pl.ds(0, nr)])
            pltpu.sync_copy(t1_h.at[pl.ds(base, nr)], av.at[pl.ds(0, nr)])
            cps = [pltpu.async_copy(t2_h.at[idxb_v.at[j]],
                                    bv.at[pl.ds(j * CHUNK, CHUNK)], sem)
                   for j in range(nj)]
            for cp in cps:
                cp.wait()
            _exp_lrelu_block(av, bv, ev, nr)

            def scale8(i8, z):
                for q in range(8):
                    i = i8 * 8 + q
                    sc = ev[pl.ds(i, 16)][0]
                    rows_v[i, pl.ds(0, 16)] = rows_v[i, pl.ds(0, 16)] * sc
                    rows_v[i, pl.ds(16, 16)] = rows_v[i, pl.ds(16, 16)] * sc
                return z
            lax.fori_loop(0, nr // 8, scale8, 0)
            cps = [pltpu.async_copy(rows_v.at[pl.ds(j * CHUNK, CHUNK)],
                                    acc_sh.at[idxb_v.at[j]], sem, add=True)
                   for j in range(nj)]
            if inc_d:
                cps += [pltpu.async_copy(ev.at[pl.ds(j * CHUNK, CHUNK)],
                                         dacc_sh.at[idxb_v.at[j]], sem,
                                         add=True)
                        for j in range(nj)]
            for cp in cps:
                cp.wait()

        def run(v_h, inc_d):
            _rr(nsb, SC_NS, sid,
                lambda sb: sblock(sb * sbs, sbs, v_h, inc_d))
            if remch:
                @pl.when(sid == SC_NS - 1)
                def _():
                    sblock(nsb * sbs, remch, v_h, inc_d)

        @pl.when(cid == 0)
        def _():
            run(hlo_h, True)

        @pl.when(cid == 1)
        def _():
            run(hhi_h, False)

        plsc.subcore_barrier()

        @pl.when(jnp.logical_and(sid == 0, cid == 0))
        def _():
            pltpu.sync_copy(acc_sh.at[pl.ds(0, nrows)], ulo_h)
            pltpu.sync_copy(dacc_sh.at[pl.ds(0, nrows)], d_h)

        @pl.when(jnp.logical_and(sid == 0, cid == 1))
        def _():
            pltpu.sync_copy(acc_sh.at[pl.ds(0, nrows)], uhi_h)

    return k(hlo, hhi, t1, t2, idx2, zero_u, zero_d)


def _pad_rows(a, mult=CHUNK, value=0):
    m = a.shape[0]
    pad = (-m) % mult
    if pad == 0:
        return a
    cfg = [(0, pad)] + [(0, 0)] * (a.ndim - 1)
    return jnp.pad(a, cfg, constant_values=value)


def _t_h0(x_r, lin1T, lin1b, attr, h0_r, r_r):
    h0 = _lrelu(x_r[...] @ lin1T[...] + lin1b[...])
    h0_r[...] = h0
    r_r[...] = h0 @ attr[...]


def _t_gate(xj_r, ea_r, rg_r, w1aT, w1bT, attl, slo_r, shi_r, e1_r):
    m = _lrelu(xj_r[...] @ w1aT[...] + ea_r[...] @ w1bT[...])
    ma = m @ attl[...]
    e = jnp.exp(_lrelu(ma + rg_r[...]))
    s = e * m
    slo_r[...] = s[:, :32]
    shi_r[...] = s[:, 32:]
    e1_r[...] = e


def _t_scale_edge(xjh_r, e_r, slo_r, shi_r, e1_r):
    e = e_r[...]
    s = e * xjh_r[...]
    slo_r[...] = s[:, :32]
    shi_r[...] = s[:, 32:]
    e1_r[...] = e


def _make_t_nodeupd(project, nxt):
    def body(*refs):
        (ulo_r, uhi_r, d_r, h_r) = refs[:4]
        i = 4
        if project:
            lin2T, bias = refs[i], refs[i + 1]
            i += 2
        else:
            bias = refs[i]
            i += 1
        gru = dict(zip(_GRU_KEYS, refs[i:i + 12]))
        i += 12
        if nxt:
            wT, asrc, adst = refs[i], refs[i + 1], refs[i + 2]
            i += 3
        outs = refs[i:]
        u = jnp.concatenate([ulo_r[...], uhi_r[...]], axis=1)
        agg = u / (d_r[...] + 1e-16)
        if project:
            agg = agg @ lin2T[...] + bias[...]
        else:
            agg = agg + bias[...]
        hcand = _elu(agg)
        gw = {kk: vv[...] for kk, vv in gru.items()}
        xc = jnp.maximum(_gru_tc(hcand, h_r[...], gw), 0.0)
        outs[0][...] = xc[:, :32]
        outs[1][...] = xc[:, 32:]
        if nxt:
            hs = xc @ wT[...]
            outs[2][...] = hs[:, :32]
            outs[3][...] = hs[:, 32:]
            outs[4][...] = hs @ asrc[...]
            outs[5][...] = hs @ adst[...]
    return body


def _t_ts_hoist(xlo_r, xhi_r, molT, asrc, hlo_r, hhi_r, t1_r):
    xc = jnp.concatenate([xlo_r[...], xhi_r[...]], axis=1)
    hs = xc @ molT[...]
    hlo_r[...] = hs[:, :32]
    hhi_r[...] = hs[:, 32:]
    t1_r[...] = hs @ asrc[...]


def _t_read_t2(ulo_r, uhi_r, molT, adst, out_r, t2_r):
    out = jnp.maximum(
        jnp.concatenate([ulo_r[...], uhi_r[...]], axis=1), 0.0)
    out_r[...] = out
    t2_r[...] = (out @ molT[...]) @ adst[...]


def _t_ts_post_t2(*refs):
    ulo_r, uhi_r, d_r, out_r = refs[:4]
    bias = refs[4]
    gru = dict(zip(_GRU_KEYS, refs[5:17]))
    molT, adst = refs[17], refs[18]
    newout_r, t2_r = refs[19], refs[20]
    u = jnp.concatenate([ulo_r[...], uhi_r[...]], axis=1)
    h = _elu(u / (d_r[...] + 1e-16) + bias[...])
    gw = {kk: vv[...] for kk, vv in gru.items()}
    out = jnp.maximum(_gru_tc(h, out_r[...], gw), 0.0)
    newout_r[...] = out
    t2_r[...] = (out @ molT[...]) @ adst[...]


_BNF = 1.0 / math.sqrt(1.0 + 1e-5)


def _t_fd(fp_r, de_r, fw1T, fb1, fg, fbb, fw2T, fb2,
          dw1T, db1, dg, dbb, dw2T, db2, f_r, d_r):
    f = jnp.maximum(fp_r[...] @ fw1T[...] + fb1[...], 0.0)
    f = f * _BNF * fg[...] + fbb[...]
    f_r[...] = jnp.maximum(f @ fw2T[...] + fb2[...], 0.0)
    d = jnp.maximum(de_r[...] @ dw1T[...] + db1[...], 0.0)
    d = d * _BNF * dg[...] + dbb[...]
    d_r[...] = jnp.maximum(d @ dw2T[...] + db2[...], 0.0)


def _t_final(out_r, f_r, d_r, lin2T, lin2b, fusT, fusb,
             toxT, toxb, regT, regb, logits_r, pct_r):
    ge = out_r[...] @ lin2T[...] + lin2b[...]
    comb = jnp.concatenate([ge, f_r[...], d_r[...]], axis=1)
    shared = jnp.maximum(comb @ fusT[...] + fusb[...], 0.0)
    logits_r[...] = shared @ toxT[...] + toxb[...]
    pct_r[...] = shared @ regT[...] + regb[...]


def kernel(x, edge_index, edge_attr, batch, fingerprints, descriptors, params):
    p = params
    n = x.shape[0]
    b = fingerprints.shape[0]
    n_layers_m1 = p['atom_lin_w'].shape[0]

    src = edge_index[0].astype(jnp.int32)
    dst = edge_index[1].astype(jnp.int32)
    batch = batch.astype(jnp.int32)

    f_mlp, d_mlp = _rowmap(
        _t_fd, [fingerprints, descriptors],
        [p['fp_w1'].T, p['fp_b1'][None, :], p['fp_bn_g'][None, :],
         p['fp_bn_b'][None, :], p['fp_w2'].T, p['fp_b2'][None, :],
         p['desc_w1'].T, p['desc_b1'][None, :], p['desc_bn_g'][None, :],
         p['desc_bn_b'][None, :], p['desc_w2'].T, p['desc_b2'][None, :]],
        [(64,), (64,)], blk_target=1024)

    h0, r = _rowmap(_t_h0, [x],
                    [p['lin1_w'].T, p['lin1_b'][None, :],
                     p['gate_att_r'][:, None]],
                    [(64,), (1,)])

    xj, rg = _sc_gather_pair(h0, r[:, 0], src, dst)
    slo, shi, e1 = _rowmap(
        _t_gate, [xj, edge_attr, rg[:, None]],
        [p['gate_lin1_w'][:, :64].T, p['gate_lin1_w'][:, 64:].T,
         p['gate_att_l'][:, None]],
        [(32,), (32,), (1,)])
    ulo, uhi, dsum = _sc_scatter_add(slo, shi, dst, e1[:, 0], n)

    def _atom_pre_aux(l):
        return [p['atom_lin_w'][l].T, p['atom_att_src'][l][:, None],
                p['atom_att_dst'][l][:, None]]

    gate_aux = [p['gate_lin2_w'].T, p['gate_bias'][None, :]]
    gru0 = _gru_aux(p['gru0_wih'], p['gru0_whh'], p['gru0_bih'], p['gru0_bhh'])
    gate_aux += [gru0[kk] for kk in _GRU_KEYS]
    xlo, xhi, hs_lo, hs_hi, a_s, a_d = _rowmap(
        _make_t_nodeupd(True, True),
        [ulo, uhi, dsum[:, None], h0], gate_aux + _atom_pre_aux(0),
        [(32,), (32,), (32,), (32,), (1,), (1,)])

    for l in range(n_layers_m1):
        ulo, uhi, dsum = _sc_layer(hs_lo, hs_hi, a_s[:, 0], a_d[:, 0],
                                   src, dst, n)
        aux = [p['atom_bias'][l][None, :]]
        grul = _gru_aux(p['atom_gru_wih'][l], p['atom_gru_whh'][l],
                        p['atom_gru_bih'][l], p['atom_gru_bhh'][l])
        aux += [grul[kk] for kk in _GRU_KEYS]
        xc_prev = jnp.concatenate([xlo, xhi], axis=1)
        last = l == n_layers_m1 - 1
        if not last:
            xlo, xhi, hs_lo, hs_hi, a_s, a_d = _rowmap(
                _make_t_nodeupd(False, True),
                [ulo, uhi, dsum[:, None], xc_prev],
                aux + _atom_pre_aux(l + 1),
                [(32,), (32,), (32,), (32,), (1,), (1,)])
        else:
            xlo, xhi = _rowmap(_make_t_nodeupd(False, False),
                               [ulo, uhi, dsum[:, None], xc_prev], aux,
                               [(32,), (32,)])

    batch_pad = _pad_rows(batch, value=b)
    ulo, uhi = _sc_scatter_add(_pad_rows(xlo), _pad_rows(xhi),
                               batch_pad, None, b)
    out, t2 = _rowmap(_t_read_t2, [ulo, uhi],
                      [p['mol_lin_w'].T, p['mol_att_dst'][:, None]],
                      [(64,), (1,)], blk_target=1024)

    hlo, hhi, t1 = _rowmap(
        _t_ts_hoist, [xlo, xhi],
        [p['mol_lin_w'].T, p['mol_att_src'][:, None]],
        [(32,), (32,), (1,)])
    hlo_p, hhi_p, t1_p = (_pad_rows(hlo), _pad_rows(hhi),
                          _pad_rows(t1[:, 0]))

    molgru = _gru_aux(p['molgru_wih'], p['molgru_whh'],
                      p['molgru_bih'], p['molgru_bhh'])
    ts_post_aux = ([p['mol_bias'][None, :]]
                   + [molgru[kk] for kk in _GRU_KEYS]
                   + [p['mol_lin_w'].T, p['mol_att_dst'][:, None]])
    for _ in range(3):
        ulo, uhi, dsum = _sc_ts_fused(hlo_p, hhi_p, t1_p, t2[:, 0],
                                      batch_pad, b)
        out, t2 = _rowmap(_t_ts_post_t2,
                          [ulo, uhi, dsum[:, None], out], ts_post_aux,
                          [(64,), (1,)], blk_target=1024)

    logits, pct = _rowmap(
        _t_final, [out, f_mlp, d_mlp],
        [p['lin2_w'].T, p['lin2_b'][None, :], p['fus_w'].T,
         p['fus_b'][None, :], p['tox_w'].T, p['tox_b'][None, :],
         p['reg_w'].T, p['reg_b'][None, :]],
        [(13,), (1,)], blk_target=1024)
    return logits, pct[:, 0]
